# Initial kernel scaffold; baseline (speedup 1.0000x reference)
import jax
import jax.numpy as jnp
from jax.experimental import pallas as pl


def kernel(x, edge_index, edge_weight, tR_indices, p, W_ih, W_hh, b_ih, b_hh, W0, W1, b1, W2, b2):
    raise NotImplementedError("write your pallas kernel here")


# speedup vs baseline: 13.1899x; 13.1899x over previous
import functools

import jax
import jax.numpy as jnp
from jax import lax
from jax.experimental import pallas as pl
from jax.experimental.pallas import tpu as pltpu
from jax.experimental.pallas import tpu_sc as plsc

f32 = jnp.float32
i32 = jnp.int32

_N = 10000
_C = 128
_H1 = 16
_E = 320000
_NPAD = 10240
_CH = 80
_NC = 2
_NS = 16
_NW = _NC * _NS
_EPW = _E // _NW
_NCH = _EPW // _CH
_RPT = _NPAD // _NS

_mesh = plsc.VectorSubcoreMesh(core_axis_name="c", subcore_axis_name="s")


def _zero_rows(buf, nrows, ncol16):
    def zb(i, _):
        for j in range(ncol16):
            buf[i, pl.ds(j * 16, 16)] = jnp.zeros((16,), f32)
        return _
    lax.fori_loop(0, nrows, zb, 0)


@functools.partial(
    pl.kernel,
    mesh=_mesh,
    compiler_params=pltpu.CompilerParams(use_tc_tiling_on_sc=False),
    out_type=jax.ShapeDtypeStruct((_NC * 3 * _NPAD, 16), f32),
    scratch_types=[
        pltpu.VMEM((_CH,), i32),
        pltpu.VMEM((_CH,), i32),
        pltpu.VMEM((_CH, 16), f32),
        pltpu.VMEM((_CH, 16), f32),
        pltpu.VMEM((64, 16), f32),
        pltpu.VMEM_SHARED((_NPAD, 16), f32),
        pltpu.VMEM_SHARED((_NPAD, 16), f32),
        pltpu.VMEM_SHARED((_NPAD, 16), f32),
    ],
)
def _sc_deg(dst_hbm, trd_hbm, wsp_hbm, out_hbm,
            dst_v, trd_v, wbuf, obuf, zbuf, acc_a, acc_b, acc_c):
    cid = lax.axis_index("c")
    sid = lax.axis_index("s")
    _zero_rows(zbuf, 64, 1)
    ones16 = jnp.ones((16,), f32)

    def ob(i, carry):
        obuf[i, :] = ones16
        return carry

    lax.fori_loop(0, _CH, ob, 0)
    for j in range(_RPT // 64):
        r0 = sid * _RPT + j * 64
        pltpu.sync_copy(zbuf, acc_a.at[pl.ds(r0, 64), :])
        pltpu.sync_copy(zbuf, acc_b.at[pl.ds(r0, 64), :])
        pltpu.sync_copy(zbuf, acc_c.at[pl.ds(r0, 64), :])
    plsc.subcore_barrier()
    base = (sid * _NC + cid) * _EPW

    def body(t, carry):
        off = pl.multiple_of(base + t * _CH, _CH)
        pltpu.sync_copy(dst_hbm.at[pl.ds(off, _CH)], dst_v)
        pltpu.sync_copy(trd_hbm.at[pl.ds(off, _CH)], trd_v)
        pltpu.sync_copy(wsp_hbm.at[pl.ds(off, _CH), :], wbuf)
        pltpu.sync_copy(wbuf, acc_a.at[dst_v], add=True)
        pltpu.sync_copy(obuf, acc_b.at[dst_v], add=True)
        pltpu.sync_copy(obuf, acc_c.at[trd_v], add=True)
        return carry

    lax.fori_loop(0, _NCH, body, 0)
    plsc.subcore_barrier()
    for j in range(_RPT // 64):
        r0 = sid * _RPT + j * 64
        pltpu.sync_copy(acc_a.at[pl.ds(r0, 64), :],
                        out_hbm.at[pl.ds(cid * 3 * _NPAD + r0, 64), :])
        pltpu.sync_copy(acc_b.at[pl.ds(r0, 64), :],
                        out_hbm.at[pl.ds(cid * 3 * _NPAD + _NPAD + r0, 64), :])
        pltpu.sync_copy(acc_c.at[pl.ds(r0, 64), :],
                        out_hbm.at[pl.ds(cid * 3 * _NPAD + 2 * _NPAD + r0, 64), :])


def _make_spmm(F, scale):
    scratch = [
        pltpu.VMEM((_CH,), i32),
        pltpu.VMEM((_CH,), i32),
        pltpu.VMEM((_CH, F), f32),
        pltpu.VMEM((64, F), f32),
        pltpu.VMEM_SHARED((_NPAD, F), f32),
        pltpu.SemaphoreType.DMA,
    ]
    if scale:
        scratch.insert(2, pltpu.VMEM((_CH, 16), f32))

    @functools.partial(
        pl.kernel,
        mesh=_mesh,
        compiler_params=pltpu.CompilerParams(use_tc_tiling_on_sc=False),
        out_type=jax.ShapeDtypeStruct((_NC * _NPAD, F), f32),
        scratch_types=scratch,
    )
    def k(*args):
        if scale:
            (src_hbm, dst_hbm, wsp_hbm, y_hbm, out_hbm,
             src_v, dst_v, wbuf, rows_v, zbuf, acc, sem) = args
        else:
            (src_hbm, dst_hbm, y_hbm, out_hbm,
             src_v, dst_v, rows_v, zbuf, acc, sem) = args
        cid = lax.axis_index("c")
        sid = lax.axis_index("s")
        _zero_rows(zbuf, 64, F // 16)
        for j in range(_RPT // 64):
            r0 = sid * _RPT + j * 64
            pltpu.sync_copy(zbuf, acc.at[pl.ds(r0, 64), :])
        plsc.subcore_barrier()
        base = (sid * _NC + cid) * _EPW

        def body(t, carry):
            off = pl.multiple_of(base + t * _CH, _CH)
            pltpu.sync_copy(src_hbm.at[pl.ds(off, _CH)], src_v)
            pltpu.sync_copy(dst_hbm.at[pl.ds(off, _CH)], dst_v)
            if scale:
                pltpu.sync_copy(wsp_hbm.at[pl.ds(off, _CH), :], wbuf)
            pltpu.async_copy(y_hbm.at[src_v], rows_v, sem).wait()
            if scale:
                def srow(i, c2):
                    wspl = wbuf[i, :]
                    for j in range(F // 16):
                        rows_v[i, pl.ds(j * 16, 16)] = (
                            rows_v[i, pl.ds(j * 16, 16)] * wspl)
                    return c2
                lax.fori_loop(0, _CH, srow, 0)
            pltpu.sync_copy(rows_v, acc.at[dst_v], add=True)
            return carry

        lax.fori_loop(0, _NCH, body, 0)
        plsc.subcore_barrier()
        for j in range(_RPT // 64):
            r0 = sid * _RPT + j * 64
            pltpu.sync_copy(acc.at[pl.ds(r0, 64), :],
                            out_hbm.at[pl.ds(cid * _NPAD + r0, 64), :])

    return k


_spmm_w128 = _make_spmm(_C, True)
_spmm_128 = _make_spmm(_C, False)

_YROWS = 1000


@functools.partial(
    pl.kernel,
    mesh=_mesh,
    compiler_params=pltpu.CompilerParams(use_tc_tiling_on_sc=False),
    out_type=jax.ShapeDtypeStruct((_NC * _NPAD, _H1), f32),
    scratch_types=[
        pltpu.VMEM((_CH,), i32),
        pltpu.VMEM((_CH,), i32),
        pltpu.VMEM((_CH, _H1), f32),
        pltpu.VMEM((_YROWS, _H1), f32),
        pltpu.VMEM((64, _H1), f32),
        pltpu.VMEM_SHARED((_N, _H1), f32),
        pltpu.VMEM_SHARED((_NPAD, _H1), f32),
        pltpu.SemaphoreType.DMA,
    ],
)
def _spmm_16(src_hbm, dst_hbm, y_hbm, out_hbm,
             src_v, dst_v, rows_v, stage_v, zbuf, y_spm, acc, sem):
    cid = lax.axis_index("c")
    sid = lax.axis_index("s")
    _zero_rows(zbuf, 64, _H1 // 16)
    for j in range(_RPT // 64):
        r0 = sid * _RPT + j * 64
        pltpu.sync_copy(zbuf, acc.at[pl.ds(r0, 64), :])
    @pl.when(sid < _N // _YROWS)
    def _stage():
        y0 = pl.multiple_of(sid * _YROWS, 8)
        pltpu.sync_copy(y_hbm.at[pl.ds(y0, _YROWS), :], stage_v)
        pltpu.sync_copy(stage_v, y_spm.at[pl.ds(y0, _YROWS), :])

    plsc.subcore_barrier()
    base = (sid * _NC + cid) * _EPW

    def body(t, carry):
        off = pl.multiple_of(base + t * _CH, _CH)
        pltpu.sync_copy(src_hbm.at[pl.ds(off, _CH)], src_v)
        pltpu.sync_copy(dst_hbm.at[pl.ds(off, _CH)], dst_v)
        pltpu.async_copy(y_spm.at[src_v], rows_v, sem).wait()
        pltpu.sync_copy(rows_v, acc.at[dst_v], add=True)
        return carry

    lax.fori_loop(0, _NCH, body, 0)
    plsc.subcore_barrier()
    for j in range(_RPT // 64):
        r0 = sid * _RPT + j * 64
        pltpu.sync_copy(acc.at[pl.ds(r0, 64), :],
                        out_hbm.at[pl.ds(cid * _NPAD + r0, 64), :])


_B = 2000
_G = _N // _B


def _tc_score(x, p2):
    def body(x_ref, p_ref, o_ref):
        pv = p_ref[...]
        nrm = jnp.sqrt(jnp.sum(pv * pv))
        o_ref[...] = jnp.tanh(
            jnp.dot(x_ref[...], pv, preferred_element_type=f32) / nrm)

    return pl.pallas_call(
        body,
        grid=(_G,),
        in_specs=[pl.BlockSpec((_B, _C), lambda i: (i, 0)),
                  pl.BlockSpec((_C, 1), lambda i: (0, 0))],
        out_specs=pl.BlockSpec((_B, 1), lambda i: (i, 0)),
        out_shape=jax.ShapeDtypeStruct((_N, 1), f32),
    )(x, p2)


def _tc_gru(xt, W0, Wih, Whh, bih, bhh):
    def body(xt_ref, w0_ref, wih_ref, whh_ref, bih_ref, bhh_ref, o_ref):
        cdims = (((1,), (1,)), ((), ()))
        gi = lax.dot_general(xt_ref[...], wih_ref[...], cdims,
                             preferred_element_type=f32) + bih_ref[...]
        gh = lax.dot_general(w0_ref[...], whh_ref[...], cdims,
                             preferred_element_type=f32) + bhh_ref[...]
        r = jax.nn.sigmoid(gi[:, :_C] + gh[:, :_C])
        z = jax.nn.sigmoid(gi[:, _C:2 * _C] + gh[:, _C:2 * _C])
        nn_ = jnp.tanh(gi[:, 2 * _C:] + r * gh[:, 2 * _C:])
        o_ref[...] = (1.0 - z) * nn_ + z * w0_ref[...]

    return pl.pallas_call(
        body,
        out_shape=jax.ShapeDtypeStruct((_C, _C), f32),
    )(xt, W0, Wih, Whh, bih, bhh)


def _tc_y1(da0, da1, x, W):
    def body(d0, d1, x_ref, w_ref, o_ref):
        dinv = lax.rsqrt(d0[...] + d1[...] + 1.0)
        o_ref[...] = dinv * jnp.dot(x_ref[...], w_ref[...],
                                    preferred_element_type=f32)

    return pl.pallas_call(
        body,
        grid=(_G,),
        in_specs=[pl.BlockSpec((_B, 1), lambda i: (i, 0)),
                  pl.BlockSpec((_B, 1), lambda i: (i, 0)),
                  pl.BlockSpec((_B, _C), lambda i: (i, 0)),
                  pl.BlockSpec((_C, _C), lambda i: (0, 0))],
        out_specs=pl.BlockSpec((_B, _C), lambda i: (i, 0)),
        out_shape=jax.ShapeDtypeStruct((_N, _C), f32),
    )(da0, da1, x, W)


def _tc_mid2(p0, p1, y1, da0, da1, db0, db1, W1):
    def body(p0r, p1r, y1r, da0r, da1r, db0r, db1r, w1r, o_ref):
        dinva = lax.rsqrt(da0r[...] + da1r[...] + 1.0)
        dinvb = lax.rsqrt(db0r[...] + db1r[...] + 1.0)
        h1 = jnp.maximum(dinva * (p0r[...] + p1r[...] + y1r[...]), 0.0)
        o_ref[...] = dinvb * jnp.dot(h1, w1r[...], preferred_element_type=f32)

    return pl.pallas_call(
        body,
        grid=(_G,),
        in_specs=[pl.BlockSpec((_B, _C), lambda i: (i, 0)),
                  pl.BlockSpec((_B, _C), lambda i: (i, 0)),
                  pl.BlockSpec((_B, _C), lambda i: (i, 0)),
                  pl.BlockSpec((_B, 1), lambda i: (i, 0)),
                  pl.BlockSpec((_B, 1), lambda i: (i, 0)),
                  pl.BlockSpec((_B, 1), lambda i: (i, 0)),
                  pl.BlockSpec((_B, 1), lambda i: (i, 0)),
                  pl.BlockSpec((_C, _H1), lambda i: (0, 0))],
        out_specs=pl.BlockSpec((_B, _H1), lambda i: (i, 0)),
        out_shape=jax.ShapeDtypeStruct((_N, _H1), f32),
    )(p0, p1, y1, da0, da1, db0, db1, W1)


def _tc_mid3(p0, p1, y2, db0, db1, dc0, dc1, W2, b1):
    def body(p0r, p1r, y2r, db0r, db1r, dc0r, dc1r, w2r, b1r, o_ref):
        dinvb = lax.rsqrt(db0r[...] + db1r[...] + 1.0)
        dinvc = lax.rsqrt(dc0r[...] + dc1r[...] + 1.0)
        h2 = jnp.maximum(
            dinvb * (p0r[...] + p1r[...] + y2r[...]) + b1r[...], 0.0)
        o_ref[...] = dinvc * jnp.dot(h2, w2r[...], preferred_element_type=f32)

    return pl.pallas_call(
        body,
        grid=(_G,),
        in_specs=[pl.BlockSpec((_B, _H1), lambda i: (i, 0)),
                  pl.BlockSpec((_B, _H1), lambda i: (i, 0)),
                  pl.BlockSpec((_B, _H1), lambda i: (i, 0)),
                  pl.BlockSpec((_B, 1), lambda i: (i, 0)),
                  pl.BlockSpec((_B, 1), lambda i: (i, 0)),
                  pl.BlockSpec((_B, 1), lambda i: (i, 0)),
                  pl.BlockSpec((_B, 1), lambda i: (i, 0)),
                  pl.BlockSpec((_H1, _C), lambda i: (0, 0)),
                  pl.BlockSpec((1, _H1), lambda i: (0, 0))],
        out_specs=pl.BlockSpec((_B, _C), lambda i: (i, 0)),
        out_shape=jax.ShapeDtypeStruct((_N, _C), f32),
    )(p0, p1, y2, db0, db1, dc0, dc1, W2, b1)


def _tc_fin(p0, p1, y3, dc0, dc1, b2):
    def body(p0r, p1r, y3r, dc0r, dc1r, b2r, o_ref):
        dinvc = lax.rsqrt(dc0r[...] + dc1r[...] + 1.0)
        o_ref[...] = dinvc * (p0r[...] + p1r[...] + y3r[...]) + b2r[...]

    return pl.pallas_call(
        body,
        grid=(_G,),
        in_specs=[pl.BlockSpec((_B, _C), lambda i: (i, 0)),
                  pl.BlockSpec((_B, _C), lambda i: (i, 0)),
                  pl.BlockSpec((_B, _C), lambda i: (i, 0)),
                  pl.BlockSpec((_B, 1), lambda i: (i, 0)),
                  pl.BlockSpec((_B, 1), lambda i: (i, 0)),
                  pl.BlockSpec((1, _C), lambda i: (0, 0))],
        out_specs=pl.BlockSpec((_B, _C), lambda i: (i, 0)),
        out_shape=jax.ShapeDtypeStruct((_N, _C), f32),
    )(p0, p1, y3, dc0, dc1, b2)


def kernel(x, edge_index, edge_weight, tR_indices, p,
           W_ih, W_hh, b_ih, b_hh, W0, W1, b1, W2, b2):
    src, dst = edge_index[0], edge_index[1]
    trs, trd = tR_indices[0], tR_indices[1]
    w_splat = jnp.broadcast_to(edge_weight[:, None], (_E, 16))

    deg = _sc_deg(dst, trd, w_splat).reshape(_NC, 3, _NPAD, 16)
    da0 = deg[0, 0, :_N, 0:1]
    da1 = deg[1, 0, :_N, 0:1]
    db0 = deg[0, 1, :_N, 0:1]
    db1 = deg[1, 1, :_N, 0:1]
    dc0 = deg[0, 2, :_N, 0:1]
    dc1 = deg[1, 2, :_N, 0:1]

    score = _tc_score(x, p.reshape(_C, 1))[:, 0]
    vals, idx = lax.top_k(score, _C)
    x_tilde = x[idx] * vals[:, None]
    Wt = _tc_gru(x_tilde, W0, W_ih, W_hh,
                 b_ih.reshape(1, -1), b_hh.reshape(1, -1))

    y1 = _tc_y1(da0, da1, x, Wt)
    p1 = _spmm_w128(src, dst, w_splat, y1).reshape(_NC, _NPAD, _C)
    y2 = _tc_mid2(p1[0, :_N], p1[1, :_N], y1, da0, da1, db0, db1, W1)
    p2 = _spmm_16(src, dst, y2).reshape(_NC, _NPAD, _H1)
    y3 = _tc_mid3(p2[0, :_N], p2[1, :_N], y2, db0, db1, dc0, dc1,
                  W2, b1.reshape(1, -1))
    p3 = _spmm_128(trs, trd, y3).reshape(_NC, _NPAD, _C)
    return _tc_fin(p3[0, :_N], p3[1, :_N], y3, dc0, dc1, b2.reshape(1, -1))


# speedup vs baseline: 24.2173x; 1.8361x over previous
import functools

import jax
import jax.numpy as jnp
from jax import lax
from jax.experimental import pallas as pl
from jax.experimental.pallas import tpu as pltpu
from jax.experimental.pallas import tpu_sc as plsc

f32 = jnp.float32
i32 = jnp.int32

_N = 10000
_C = 128
_H1 = 16
_E = 320000
_NPAD = 10240
_CH = 128
_NC = 2
_NS = 16
_NW = _NC * _NS
_EPW = 10240
_EPAD = _EPW * _NW
_NCH = _EPW // _CH
_NQ = _NCH // 2
_RPT = _NPAD // _NS

_mesh = plsc.VectorSubcoreMesh(core_axis_name="c", subcore_axis_name="s")
_sc_params = pltpu.CompilerParams(use_tc_tiling_on_sc=False)


def _zero_rows(buf, nrows, ncol16):
    def zb(i, _):
        for j in range(ncol16):
            buf[i, pl.ds(j * 16, 16)] = jnp.zeros((16,), f32)
        return _
    lax.fori_loop(0, nrows, zb, 0)


def _coff(base, t):
    return pl.multiple_of(base + t * _CH, _CH)


@functools.partial(
    pl.kernel,
    mesh=_mesh,
    compiler_params=_sc_params,
    out_type=jax.ShapeDtypeStruct((_NC * 3 * _NPAD, 16), f32),
    scratch_types=[
        pltpu.VMEM((_CH,), i32), pltpu.VMEM((_CH,), i32),
        pltpu.VMEM((_CH,), i32), pltpu.VMEM((_CH,), i32),
        pltpu.VMEM((_CH, 16), f32), pltpu.VMEM((_CH, 16), f32),
        pltpu.VMEM((_CH, 16), f32),
        pltpu.VMEM((64, 16), f32),
        pltpu.VMEM_SHARED((_NPAD, 16), f32),
        pltpu.VMEM_SHARED((_NPAD, 16), f32),
        pltpu.VMEM_SHARED((_NPAD, 16), f32),
        pltpu.SemaphoreType.DMA, pltpu.SemaphoreType.DMA,
    ],
)
def _sc_deg(dst_hbm, trd_hbm, wsp_hbm, out_hbm,
            dstA, dstB, trdA, trdB, wbA, wbB, obuf, zbuf,
            acc_a, acc_b, acc_c, semA, semB):
    cid = lax.axis_index("c")
    sid = lax.axis_index("s")
    _zero_rows(zbuf, 64, 1)
    ones16 = jnp.ones((16,), f32)

    def ob(i, carry):
        obuf[i, :] = ones16
        return carry

    lax.fori_loop(0, _CH, ob, 0)
    for j in range(_RPT // 64):
        r0 = sid * _RPT + j * 64
        pltpu.sync_copy(zbuf, acc_a.at[pl.ds(r0, 64), :])
        pltpu.sync_copy(zbuf, acc_b.at[pl.ds(r0, 64), :])
        pltpu.sync_copy(zbuf, acc_c.at[pl.ds(r0, 64), :])
    plsc.subcore_barrier()
    base = (sid * _NC + cid) * _EPW

    def loads(t, d_v, t_v, w_v, sem):
        off = _coff(base, t)
        pltpu.async_copy(dst_hbm.at[pl.ds(off, _CH)], d_v, sem)
        pltpu.async_copy(trd_hbm.at[pl.ds(off, _CH)], t_v, sem)
        pltpu.async_copy(wsp_hbm.at[pl.ds(off, _CH), :], w_v, sem)

    def drain(t, d_v, t_v, w_v, sem):
        off = _coff(base, t)
        pltpu.make_async_copy(dst_hbm.at[pl.ds(off, _CH)], d_v, sem).wait()
        pltpu.make_async_copy(trd_hbm.at[pl.ds(off, _CH)], t_v, sem).wait()
        pltpu.make_async_copy(wsp_hbm.at[pl.ds(off, _CH), :], w_v, sem).wait()

    def scatters(d_v, t_v, w_v):
        pltpu.sync_copy(w_v, acc_a.at[d_v], add=True)
        pltpu.sync_copy(obuf, acc_b.at[d_v], add=True)
        pltpu.sync_copy(obuf, acc_c.at[t_v], add=True)

    loads(0, dstA, trdA, wbA, semA)
    loads(1, dstB, trdB, wbB, semB)

    def body(q, carry):
        t0 = 2 * q
        drain(t0, dstA, trdA, wbA, semA)
        scatters(dstA, trdA, wbA)

        @pl.when(q + 1 < _NQ)
        def _():
            loads(t0 + 2, dstA, trdA, wbA, semA)

        drain(t0 + 1, dstB, trdB, wbB, semB)
        scatters(dstB, trdB, wbB)

        @pl.when(q + 1 < _NQ)
        def _():
            loads(t0 + 3, dstB, trdB, wbB, semB)

        return carry

    lax.fori_loop(0, _NQ, body, 0)
    plsc.subcore_barrier()
    for j in range(_RPT // 64):
        r0 = sid * _RPT + j * 64
        for a, acc in enumerate((acc_a, acc_b, acc_c)):
            pltpu.sync_copy(acc.at[pl.ds(r0, 64), :],
                            out_hbm.at[pl.ds((cid * 3 + a) * _NPAD + r0, 64), :])


def _make_spmm(F, scale, small):
    scratch = [
        pltpu.VMEM((_CH,), i32), pltpu.VMEM((_CH,), i32),
        pltpu.VMEM((_CH,), i32), pltpu.VMEM((_CH,), i32),
        pltpu.VMEM((_CH, F), f32), pltpu.VMEM((_CH, F), f32),
        pltpu.VMEM((64, F), f32),
        pltpu.VMEM_SHARED((_NPAD, F), f32),
        pltpu.SemaphoreType.DMA, pltpu.SemaphoreType.DMA,
        pltpu.SemaphoreType.DMA, pltpu.SemaphoreType.DMA,
    ]
    if scale:
        scratch = [pltpu.VMEM((_CH, 16), f32),
                   pltpu.VMEM((_CH, 16), f32)] + scratch
    if small:
        scratch = [pltpu.VMEM((1000, F), f32),
                   pltpu.VMEM_SHARED((_N, F), f32)] + scratch

    @functools.partial(
        pl.kernel,
        mesh=_mesh,
        compiler_params=_sc_params,
        out_type=jax.ShapeDtypeStruct((_NC * _NPAD, F), f32),
        scratch_types=scratch,
    )
    def k(*args):
        src_hbm, dst_hbm = args[0], args[1]
        a = 2
        wsp_hbm = args[a] if scale else None
        a += 1 if scale else 0
        y_hbm, out_hbm = args[a], args[a + 1]
        a += 2
        if small:
            stage_v, y_spm = args[a], args[a + 1]
            a += 2
        if scale:
            wbA, wbB = args[a], args[a + 1]
            a += 2
        (srcA, srcB, dstA, dstB, rowsA, rowsB, zbuf, acc,
         semIA, semIB, semGA, semGB) = args[a:]

        cid = lax.axis_index("c")
        sid = lax.axis_index("s")
        _zero_rows(zbuf, 64, F // 16)
        for j in range(_RPT // 64):
            r0 = sid * _RPT + j * 64
            pltpu.sync_copy(zbuf, acc.at[pl.ds(r0, 64), :])
        if small:
            @pl.when(sid < _N // 1000)
            def _stage():
                y0 = pl.multiple_of(sid * 1000, 8)
                pltpu.sync_copy(y_hbm.at[pl.ds(y0, 1000), :], stage_v)
                pltpu.sync_copy(stage_v, y_spm.at[pl.ds(y0, 1000), :])
            ysrc = y_spm
        else:
            ysrc = y_hbm
        plsc.subcore_barrier()
        base = (sid * _NC + cid) * _EPW

        def loads(t, s_v, d_v, w_v, sem):
            off = _coff(base, t)
            pltpu.async_copy(src_hbm.at[pl.ds(off, _CH)], s_v, sem)
            pltpu.async_copy(dst_hbm.at[pl.ds(off, _CH)], d_v, sem)
            if scale:
                pltpu.async_copy(wsp_hbm.at[pl.ds(off, _CH), :], w_v, sem)

        def drain_loads(t, s_v, d_v, w_v, sem):
            off = _coff(base, t)
            pltpu.make_async_copy(src_hbm.at[pl.ds(off, _CH)], s_v, sem).wait()
            pltpu.make_async_copy(dst_hbm.at[pl.ds(off, _CH)], d_v, sem).wait()
            if scale:
                pltpu.make_async_copy(
                    wsp_hbm.at[pl.ds(off, _CH), :], w_v, sem).wait()

        def do_scale(rows_v, w_v):
            if scale:
                def srow(i, c2):
                    wspl = w_v[i, :]
                    for j in range(F // 16):
                        rows_v[i, pl.ds(j * 16, 16)] = (
                            rows_v[i, pl.ds(j * 16, 16)] * wspl)
                    return c2
                lax.fori_loop(0, _CH, srow, 0)

        wbA_ = wbA if scale else None
        wbB_ = wbB if scale else None
        loads(0, srcA, dstA, wbA_, semIA)
        loads(1, srcB, dstB, wbB_, semIB)
        drain_loads(0, srcA, dstA, wbA_, semIA)
        pltpu.async_copy(ysrc.at[srcA], rowsA, semGA)

        def body(q, carry):
            t0 = 2 * q
            drain_loads(t0 + 1, srcB, dstB, wbB_, semIB)
            pltpu.async_copy(ysrc.at[srcB], rowsB, semGB)
            pltpu.make_async_copy(ysrc.at[srcA], rowsA, semGA).wait()
            do_scale(rowsA, wbA_)
            pltpu.sync_copy(rowsA, acc.at[dstA], add=True)

            @pl.when(q + 1 < _NQ)
            def _():
                loads(t0 + 2, srcA, dstA, wbA_, semIA)

            pltpu.make_async_copy(ysrc.at[srcB], rowsB, semGB).wait()
            do_scale(rowsB, wbB_)

            @pl.when(q + 1 < _NQ)
            def _():
                drain_loads(t0 + 2, srcA, dstA, wbA_, semIA)
                pltpu.async_copy(ysrc.at[srcA], rowsA, semGA)

            pltpu.sync_copy(rowsB, acc.at[dstB], add=True)

            @pl.when(q + 1 < _NQ)
            def _():
                loads(t0 + 3, srcB, dstB, wbB_, semIB)

            return carry

        lax.fori_loop(0, _NQ, body, 0)
        plsc.subcore_barrier()
        for j in range(_RPT // 64):
            r0 = sid * _RPT + j * 64
            pltpu.sync_copy(acc.at[pl.ds(r0, 64), :],
                            out_hbm.at[pl.ds(cid * _NPAD + r0, 64), :])

    return k


_spmm_w128 = _make_spmm(_C, True, False)
_spmm_128 = _make_spmm(_C, False, False)
_spmm_16 = _make_spmm(_H1, False, True)


_B = 2000
_G = _N // _B


def _tc_score(x, p2):
    def body(x_ref, p_ref, o_ref):
        pv = p_ref[...]
        nrm = jnp.sqrt(jnp.sum(pv * pv))
        o_ref[...] = jnp.tanh(
            jnp.dot(x_ref[...], pv, preferred_element_type=f32) / nrm)

    return pl.pallas_call(
        body,
        grid=(_G,),
        in_specs=[pl.BlockSpec((_B, _C), lambda i: (i, 0)),
                  pl.BlockSpec((_C, 1), lambda i: (0, 0))],
        out_specs=pl.BlockSpec((_B, 1), lambda i: (i, 0)),
        out_shape=jax.ShapeDtypeStruct((_N, 1), f32),
    )(x, p2)


def _tc_gru(xt, W0, Wih, Whh, bih, bhh):
    def body(xt_ref, w0_ref, wih_ref, whh_ref, bih_ref, bhh_ref, o_ref):
        cdims = (((1,), (1,)), ((), ()))
        gi = lax.dot_general(xt_ref[...], wih_ref[...], cdims,
                             preferred_element_type=f32) + bih_ref[...]
        gh = lax.dot_general(w0_ref[...], whh_ref[...], cdims,
                             preferred_element_type=f32) + bhh_ref[...]
        r = jax.nn.sigmoid(gi[:, :_C] + gh[:, :_C])
        z = jax.nn.sigmoid(gi[:, _C:2 * _C] + gh[:, _C:2 * _C])
        nn_ = jnp.tanh(gi[:, 2 * _C:] + r * gh[:, 2 * _C:])
        o_ref[...] = (1.0 - z) * nn_ + z * w0_ref[...]

    return pl.pallas_call(
        body,
        out_shape=jax.ShapeDtypeStruct((_C, _C), f32),
    )(xt, W0, Wih, Whh, bih, bhh)


def _tc_y1(da0, da1, x, W):
    def body(d0, d1, x_ref, w_ref, o_ref):
        dinv = lax.rsqrt(d0[...] + d1[...] + 1.0)
        o_ref[...] = dinv * jnp.dot(x_ref[...], w_ref[...],
                                    preferred_element_type=f32)

    return pl.pallas_call(
        body,
        grid=(_G,),
        in_specs=[pl.BlockSpec((_B, 1), lambda i: (i, 0)),
                  pl.BlockSpec((_B, 1), lambda i: (i, 0)),
                  pl.BlockSpec((_B, _C), lambda i: (i, 0)),
                  pl.BlockSpec((_C, _C), lambda i: (0, 0))],
        out_specs=pl.BlockSpec((_B, _C), lambda i: (i, 0)),
        out_shape=jax.ShapeDtypeStruct((_N, _C), f32),
    )(da0, da1, x, W)


def _tc_mid2(p0, p1, y1, da0, da1, db0, db1, W1):
    def body(p0r, p1r, y1r, da0r, da1r, db0r, db1r, w1r, o_ref):
        dinva = lax.rsqrt(da0r[...] + da1r[...] + 1.0)
        dinvb = lax.rsqrt(db0r[...] + db1r[...] + 1.0)
        h1 = jnp.maximum(dinva * (p0r[...] + p1r[...] + y1r[...]), 0.0)
        o_ref[...] = dinvb * jnp.dot(h1, w1r[...], preferred_element_type=f32)

    return pl.pallas_call(
        body,
        grid=(_G,),
        in_specs=[pl.BlockSpec((_B, _C), lambda i: (i, 0)),
                  pl.BlockSpec((_B, _C), lambda i: (i, 0)),
                  pl.BlockSpec((_B, _C), lambda i: (i, 0)),
                  pl.BlockSpec((_B, 1), lambda i: (i, 0)),
                  pl.BlockSpec((_B, 1), lambda i: (i, 0)),
                  pl.BlockSpec((_B, 1), lambda i: (i, 0)),
                  pl.BlockSpec((_B, 1), lambda i: (i, 0)),
                  pl.BlockSpec((_C, _H1), lambda i: (0, 0))],
        out_specs=pl.BlockSpec((_B, _H1), lambda i: (i, 0)),
        out_shape=jax.ShapeDtypeStruct((_N, _H1), f32),
    )(p0, p1, y1, da0, da1, db0, db1, W1)


def _tc_mid3(p0, p1, y2, db0, db1, dc0, dc1, W2, b1):
    def body(p0r, p1r, y2r, db0r, db1r, dc0r, dc1r, w2r, b1r, o_ref):
        dinvb = lax.rsqrt(db0r[...] + db1r[...] + 1.0)
        dinvc = lax.rsqrt(dc0r[...] + dc1r[...] + 1.0)
        h2 = jnp.maximum(
            dinvb * (p0r[...] + p1r[...] + y2r[...]) + b1r[...], 0.0)
        o_ref[...] = dinvc * jnp.dot(h2, w2r[...], preferred_element_type=f32)

    return pl.pallas_call(
        body,
        grid=(_G,),
        in_specs=[pl.BlockSpec((_B, _H1), lambda i: (i, 0)),
                  pl.BlockSpec((_B, _H1), lambda i: (i, 0)),
                  pl.BlockSpec((_B, _H1), lambda i: (i, 0)),
                  pl.BlockSpec((_B, 1), lambda i: (i, 0)),
                  pl.BlockSpec((_B, 1), lambda i: (i, 0)),
                  pl.BlockSpec((_B, 1), lambda i: (i, 0)),
                  pl.BlockSpec((_B, 1), lambda i: (i, 0)),
                  pl.BlockSpec((_H1, _C), lambda i: (0, 0)),
                  pl.BlockSpec((1, _H1), lambda i: (0, 0))],
        out_specs=pl.BlockSpec((_B, _C), lambda i: (i, 0)),
        out_shape=jax.ShapeDtypeStruct((_N, _C), f32),
    )(p0, p1, y2, db0, db1, dc0, dc1, W2, b1)


def _tc_fin(p0, p1, y3, dc0, dc1, b2):
    def body(p0r, p1r, y3r, dc0r, dc1r, b2r, o_ref):
        dinvc = lax.rsqrt(dc0r[...] + dc1r[...] + 1.0)
        o_ref[...] = dinvc * (p0r[...] + p1r[...] + y3r[...]) + b2r[...]

    return pl.pallas_call(
        body,
        grid=(_G,),
        in_specs=[pl.BlockSpec((_B, _C), lambda i: (i, 0)),
                  pl.BlockSpec((_B, _C), lambda i: (i, 0)),
                  pl.BlockSpec((_B, _C), lambda i: (i, 0)),
                  pl.BlockSpec((_B, 1), lambda i: (i, 0)),
                  pl.BlockSpec((_B, 1), lambda i: (i, 0)),
                  pl.BlockSpec((1, _C), lambda i: (0, 0))],
        out_specs=pl.BlockSpec((_B, _C), lambda i: (i, 0)),
        out_shape=jax.ShapeDtypeStruct((_N, _C), f32),
    )(p0, p1, y3, dc0, dc1, b2)


def kernel(x, edge_index, edge_weight, tR_indices, p,
           W_ih, W_hh, b_ih, b_hh, W0, W1, b1, W2, b2):
    npad = _EPAD - _E
    pad_src = (jnp.arange(npad, dtype=i32) * 53) % _N
    pad_dst = _N + (jnp.arange(npad, dtype=i32) % (_NPAD - _N))
    src = jnp.concatenate([edge_index[0].astype(i32), pad_src])
    dst = jnp.concatenate([edge_index[1].astype(i32), pad_dst])
    trs = jnp.concatenate([tR_indices[0].astype(i32), pad_src])
    trd = jnp.concatenate([tR_indices[1].astype(i32), pad_dst])
    wp = jnp.concatenate([edge_weight, jnp.zeros((npad,), f32)])
    w_splat = jnp.broadcast_to(wp[:, None], (_EPAD, 16))

    deg = _sc_deg(dst, trd, w_splat).reshape(_NC, 3, _NPAD, 16)
    da0 = deg[0, 0, :_N, 0:1]
    da1 = deg[1, 0, :_N, 0:1]
    db0 = deg[0, 1, :_N, 0:1]
    db1 = deg[1, 1, :_N, 0:1]
    dc0 = deg[0, 2, :_N, 0:1]
    dc1 = deg[1, 2, :_N, 0:1]

    score = _tc_score(x, p.reshape(_C, 1))[:, 0]
    vals, idx = lax.top_k(score, _C)
    x_tilde = x[idx] * vals[:, None]
    Wt = _tc_gru(x_tilde, W0, W_ih, W_hh,
                 b_ih.reshape(1, -1), b_hh.reshape(1, -1))

    y1 = _tc_y1(da0, da1, x, Wt)
    p1 = _spmm_w128(src, dst, w_splat, y1).reshape(_NC, _NPAD, _C)
    y2 = _tc_mid2(p1[0, :_N], p1[1, :_N], y1, da0, da1, db0, db1, W1)
    p2 = _spmm_16(src, dst, y2).reshape(_NC, _NPAD, _H1)
    y3 = _tc_mid3(p2[0, :_N], p2[1, :_N], y2, db0, db1, dc0, dc1,
                  W2, b1.reshape(1, -1))
    p3 = _spmm_128(trs, trd, y3).reshape(_NC, _NPAD, _C)
    return _tc_fin(p3[0, :_N], p3[1, :_N], y3, dc0, dc1, b2.reshape(1, -1))


# speedup vs baseline: 24.2193x; 1.0001x over previous
import functools

import jax
import jax.numpy as jnp
from jax import lax
from jax.experimental import pallas as pl
from jax.experimental.pallas import tpu as pltpu
from jax.experimental.pallas import tpu_sc as plsc

f32 = jnp.float32
i32 = jnp.int32

_N = 10000
_C = 128
_H1 = 16
_E = 320000
_NPAD = 10240
_CH = 128
_NC = 2
_NS = 16
_NW = _NC * _NS
_EPW = 10240
_EPAD = _EPW * _NW
_NCH = _EPW // _CH
_NQ = _NCH // 2
_RPT = _NPAD // _NS

_mesh = plsc.VectorSubcoreMesh(core_axis_name="c", subcore_axis_name="s")
_sc_params = pltpu.CompilerParams(use_tc_tiling_on_sc=False)


def _zero_rows(buf, nrows, ncol16):
    def zb(i, _):
        for j in range(ncol16):
            buf[i, pl.ds(j * 16, 16)] = jnp.zeros((16,), f32)
        return _
    lax.fori_loop(0, nrows, zb, 0)


def _coff(base, t):
    return pl.multiple_of(base + t * _CH, _CH)


@functools.partial(
    pl.kernel,
    mesh=_mesh,
    compiler_params=_sc_params,
    out_type=jax.ShapeDtypeStruct((_NC * 3 * _NPAD, 16), f32),
    scratch_types=[
        pltpu.VMEM((_CH,), i32), pltpu.VMEM((_CH,), i32),
        pltpu.VMEM((_CH,), i32), pltpu.VMEM((_CH,), i32),
        pltpu.VMEM((_CH, 16), f32), pltpu.VMEM((_CH, 16), f32),
        pltpu.VMEM((_CH, 16), f32),
        pltpu.VMEM((64, 16), f32),
        pltpu.VMEM_SHARED((_NPAD, 16), f32),
        pltpu.VMEM_SHARED((_NPAD, 16), f32),
        pltpu.VMEM_SHARED((_NPAD, 16), f32),
        pltpu.SemaphoreType.DMA, pltpu.SemaphoreType.DMA,
    ],
)
def _sc_deg(dst_hbm, trd_hbm, wsp_hbm, out_hbm,
            dstA, dstB, trdA, trdB, wbA, wbB, obuf, zbuf,
            acc_a, acc_b, acc_c, semA, semB):
    cid = lax.axis_index("c")
    sid = lax.axis_index("s")
    _zero_rows(zbuf, 64, 1)
    ones16 = jnp.ones((16,), f32)

    def ob(i, carry):
        obuf[i, :] = ones16
        return carry

    lax.fori_loop(0, _CH, ob, 0)
    for j in range(_RPT // 64):
        r0 = sid * _RPT + j * 64
        pltpu.sync_copy(zbuf, acc_a.at[pl.ds(r0, 64), :])
        pltpu.sync_copy(zbuf, acc_b.at[pl.ds(r0, 64), :])
        pltpu.sync_copy(zbuf, acc_c.at[pl.ds(r0, 64), :])
    plsc.subcore_barrier()
    base = (sid * _NC + cid) * _EPW

    def loads(t, d_v, t_v, w_v, sem):
        off = _coff(base, t)
        pltpu.async_copy(dst_hbm.at[pl.ds(off, _CH)], d_v, sem)
        pltpu.async_copy(trd_hbm.at[pl.ds(off, _CH)], t_v, sem)
        pltpu.async_copy(wsp_hbm.at[pl.ds(off, _CH), :], w_v, sem)

    def drain(t, d_v, t_v, w_v, sem):
        off = _coff(base, t)
        pltpu.make_async_copy(dst_hbm.at[pl.ds(off, _CH)], d_v, sem).wait()
        pltpu.make_async_copy(trd_hbm.at[pl.ds(off, _CH)], t_v, sem).wait()
        pltpu.make_async_copy(wsp_hbm.at[pl.ds(off, _CH), :], w_v, sem).wait()

    def scatters(d_v, t_v, w_v):
        pltpu.sync_copy(w_v, acc_a.at[d_v], add=True)
        pltpu.sync_copy(obuf, acc_b.at[d_v], add=True)
        pltpu.sync_copy(obuf, acc_c.at[t_v], add=True)

    loads(0, dstA, trdA, wbA, semA)
    loads(1, dstB, trdB, wbB, semB)

    def body(q, carry):
        t0 = 2 * q
        drain(t0, dstA, trdA, wbA, semA)
        scatters(dstA, trdA, wbA)

        @pl.when(q + 1 < _NQ)
        def _():
            loads(t0 + 2, dstA, trdA, wbA, semA)

        drain(t0 + 1, dstB, trdB, wbB, semB)
        scatters(dstB, trdB, wbB)

        @pl.when(q + 1 < _NQ)
        def _():
            loads(t0 + 3, dstB, trdB, wbB, semB)

        return carry

    lax.fori_loop(0, _NQ, body, 0)
    plsc.subcore_barrier()
    for j in range(_RPT // 64):
        r0 = sid * _RPT + j * 64
        for a, acc in enumerate((acc_a, acc_b, acc_c)):
            pltpu.sync_copy(acc.at[pl.ds(r0, 64), :],
                            out_hbm.at[pl.ds((cid * 3 + a) * _NPAD + r0, 64), :])


def _make_spmm(F, scale, small):
    scratch = [
        pltpu.VMEM((_CH,), i32), pltpu.VMEM((_CH,), i32),
        pltpu.VMEM((_CH,), i32), pltpu.VMEM((_CH,), i32),
        pltpu.VMEM((_CH, F), f32), pltpu.VMEM((_CH, F), f32),
        pltpu.VMEM((64, F), f32),
        pltpu.VMEM_SHARED((_NPAD, F), f32),
        pltpu.SemaphoreType.DMA, pltpu.SemaphoreType.DMA,
        pltpu.SemaphoreType.DMA, pltpu.SemaphoreType.DMA,
    ]
    if scale:
        scratch = [pltpu.VMEM((_CH, 16), f32),
                   pltpu.VMEM((_CH, 16), f32)] + scratch
    if small:
        scratch = [pltpu.VMEM((1000, F), f32),
                   pltpu.VMEM_SHARED((_N, F), f32)] + scratch

    @functools.partial(
        pl.kernel,
        mesh=_mesh,
        compiler_params=_sc_params,
        out_type=jax.ShapeDtypeStruct((_NC * _NPAD, F), f32),
        scratch_types=scratch,
    )
    def k(*args):
        src_hbm, dst_hbm = args[0], args[1]
        a = 2
        wsp_hbm = args[a] if scale else None
        a += 1 if scale else 0
        y_hbm, out_hbm = args[a], args[a + 1]
        a += 2
        if small:
            stage_v, y_spm = args[a], args[a + 1]
            a += 2
        if scale:
            wbA, wbB = args[a], args[a + 1]
            a += 2
        (srcA, srcB, dstA, dstB, rowsA, rowsB, zbuf, acc,
         semIA, semIB, semGA, semGB) = args[a:]

        cid = lax.axis_index("c")
        sid = lax.axis_index("s")
        _zero_rows(zbuf, 64, F // 16)
        for j in range(_RPT // 64):
            r0 = sid * _RPT + j * 64
            pltpu.sync_copy(zbuf, acc.at[pl.ds(r0, 64), :])
        if small:
            @pl.when(sid < _N // 1000)
            def _stage():
                y0 = pl.multiple_of(sid * 1000, 8)
                pltpu.sync_copy(y_hbm.at[pl.ds(y0, 1000), :], stage_v)
                pltpu.sync_copy(stage_v, y_spm.at[pl.ds(y0, 1000), :])
            ysrc = y_spm
        else:
            ysrc = y_hbm
        plsc.subcore_barrier()
        base = (sid * _NC + cid) * _EPW

        def loads(t, s_v, d_v, w_v, sem):
            off = _coff(base, t)
            pltpu.async_copy(src_hbm.at[pl.ds(off, _CH)], s_v, sem)
            pltpu.async_copy(dst_hbm.at[pl.ds(off, _CH)], d_v, sem)
            if scale:
                pltpu.async_copy(wsp_hbm.at[pl.ds(off, _CH), :], w_v, sem)

        def drain_loads(t, s_v, d_v, w_v, sem):
            off = _coff(base, t)
            pltpu.make_async_copy(src_hbm.at[pl.ds(off, _CH)], s_v, sem).wait()
            pltpu.make_async_copy(dst_hbm.at[pl.ds(off, _CH)], d_v, sem).wait()
            if scale:
                pltpu.make_async_copy(
                    wsp_hbm.at[pl.ds(off, _CH), :], w_v, sem).wait()

        def do_scale(rows_v, w_v):
            if scale:
                def srow(i, c2):
                    wspl = w_v[i, :]
                    for j in range(F // 16):
                        rows_v[i, pl.ds(j * 16, 16)] = (
                            rows_v[i, pl.ds(j * 16, 16)] * wspl)
                    return c2
                lax.fori_loop(0, _CH, srow, 0)

        wbA_ = wbA if scale else None
        wbB_ = wbB if scale else None
        loads(0, srcA, dstA, wbA_, semIA)
        loads(1, srcB, dstB, wbB_, semIB)
        drain_loads(0, srcA, dstA, wbA_, semIA)
        pltpu.async_copy(ysrc.at[srcA], rowsA, semGA)

        def body(q, carry):
            t0 = 2 * q
            drain_loads(t0 + 1, srcB, dstB, wbB_, semIB)
            pltpu.async_copy(ysrc.at[srcB], rowsB, semGB)
            pltpu.make_async_copy(ysrc.at[srcA], rowsA, semGA).wait()
            do_scale(rowsA, wbA_)
            pltpu.sync_copy(rowsA, acc.at[dstA], add=True)

            @pl.when(q + 1 < _NQ)
            def _():
                loads(t0 + 2, srcA, dstA, wbA_, semIA)

            pltpu.make_async_copy(ysrc.at[srcB], rowsB, semGB).wait()
            do_scale(rowsB, wbB_)

            @pl.when(q + 1 < _NQ)
            def _():
                drain_loads(t0 + 2, srcA, dstA, wbA_, semIA)
                pltpu.async_copy(ysrc.at[srcA], rowsA, semGA)

            pltpu.sync_copy(rowsB, acc.at[dstB], add=True)

            @pl.when(q + 1 < _NQ)
            def _():
                loads(t0 + 3, srcB, dstB, wbB_, semIB)

            return carry

        lax.fori_loop(0, _NQ, body, 0)
        plsc.subcore_barrier()
        for j in range(_RPT // 64):
            r0 = sid * _RPT + j * 64
            pltpu.sync_copy(acc.at[pl.ds(r0, 64), :],
                            out_hbm.at[pl.ds(cid * _NPAD + r0, 64), :])

    return k


_spmm_w128 = _make_spmm(_C, True, False)
_spmm_128 = _make_spmm(_C, False, False)
_spmm_16 = _make_spmm(_H1, False, True)


_B = 2000
_G = _N // _B


def _tc_score(x, p2):
    def body(x_ref, p_ref, o_ref):
        pv = p_ref[...]
        nrm = jnp.sqrt(jnp.sum(pv * pv))
        o_ref[...] = jnp.tanh(
            jnp.dot(x_ref[...], pv, preferred_element_type=f32) / nrm)

    return pl.pallas_call(
        body,
        grid=(_G,),
        in_specs=[pl.BlockSpec((_B, _C), lambda i: (i, 0)),
                  pl.BlockSpec((_C, 1), lambda i: (0, 0))],
        out_specs=pl.BlockSpec((_B, 1), lambda i: (i, 0)),
        out_shape=jax.ShapeDtypeStruct((_N, 1), f32),
    )(x, p2)


def _tc_gru(xt, W0, Wih, Whh, bih, bhh):
    def body(xt_ref, w0_ref, wih_ref, whh_ref, bih_ref, bhh_ref, o_ref):
        cdims = (((1,), (1,)), ((), ()))
        gi = lax.dot_general(xt_ref[...], wih_ref[...], cdims,
                             preferred_element_type=f32) + bih_ref[...]
        gh = lax.dot_general(w0_ref[...], whh_ref[...], cdims,
                             preferred_element_type=f32) + bhh_ref[...]
        r = jax.nn.sigmoid(gi[:, :_C] + gh[:, :_C])
        z = jax.nn.sigmoid(gi[:, _C:2 * _C] + gh[:, _C:2 * _C])
        nn_ = jnp.tanh(gi[:, 2 * _C:] + r * gh[:, 2 * _C:])
        o_ref[...] = (1.0 - z) * nn_ + z * w0_ref[...]

    return pl.pallas_call(
        body,
        out_shape=jax.ShapeDtypeStruct((_C, _C), f32),
    )(xt, W0, Wih, Whh, bih, bhh)


def _tc_y1(da0, da1, x, W):
    def body(d0, d1, x_ref, w_ref, o_ref):
        dinv = lax.rsqrt(d0[...] + d1[...] + 1.0)
        o_ref[...] = dinv * jnp.dot(x_ref[...], w_ref[...],
                                    preferred_element_type=f32)

    return pl.pallas_call(
        body,
        grid=(_G,),
        in_specs=[pl.BlockSpec((_B, 1), lambda i: (i, 0)),
                  pl.BlockSpec((_B, 1), lambda i: (i, 0)),
                  pl.BlockSpec((_B, _C), lambda i: (i, 0)),
                  pl.BlockSpec((_C, _C), lambda i: (0, 0))],
        out_specs=pl.BlockSpec((_B, _C), lambda i: (i, 0)),
        out_shape=jax.ShapeDtypeStruct((_N, _C), f32),
    )(da0, da1, x, W)


def _tc_mid2(p0, p1, y1, da0, da1, db0, db1, W1):
    def body(p0r, p1r, y1r, da0r, da1r, db0r, db1r, w1r, o_ref):
        dinva = lax.rsqrt(da0r[...] + da1r[...] + 1.0)
        dinvb = lax.rsqrt(db0r[...] + db1r[...] + 1.0)
        h1 = jnp.maximum(dinva * (p0r[...] + p1r[...] + y1r[...]), 0.0)
        o_ref[...] = dinvb * jnp.dot(h1, w1r[...], preferred_element_type=f32)

    return pl.pallas_call(
        body,
        grid=(_G,),
        in_specs=[pl.BlockSpec((_B, _C), lambda i: (i, 0)),
                  pl.BlockSpec((_B, _C), lambda i: (i, 0)),
                  pl.BlockSpec((_B, _C), lambda i: (i, 0)),
                  pl.BlockSpec((_B, 1), lambda i: (i, 0)),
                  pl.BlockSpec((_B, 1), lambda i: (i, 0)),
                  pl.BlockSpec((_B, 1), lambda i: (i, 0)),
                  pl.BlockSpec((_B, 1), lambda i: (i, 0)),
                  pl.BlockSpec((_C, _H1), lambda i: (0, 0))],
        out_specs=pl.BlockSpec((_B, _H1), lambda i: (i, 0)),
        out_shape=jax.ShapeDtypeStruct((_N, _H1), f32),
    )(p0, p1, y1, da0, da1, db0, db1, W1)


def _tc_mid3(p0, p1, y2, db0, db1, dc0, dc1, W2, b1):
    def body(p0r, p1r, y2r, db0r, db1r, dc0r, dc1r, w2r, b1r, o_ref):
        dinvb = lax.rsqrt(db0r[...] + db1r[...] + 1.0)
        dinvc = lax.rsqrt(dc0r[...] + dc1r[...] + 1.0)
        h2 = jnp.maximum(
            dinvb * (p0r[...] + p1r[...] + y2r[...]) + b1r[...], 0.0)
        o_ref[...] = dinvc * jnp.dot(h2, w2r[...], preferred_element_type=f32)

    return pl.pallas_call(
        body,
        grid=(_G,),
        in_specs=[pl.BlockSpec((_B, _H1), lambda i: (i, 0)),
                  pl.BlockSpec((_B, _H1), lambda i: (i, 0)),
                  pl.BlockSpec((_B, _H1), lambda i: (i, 0)),
                  pl.BlockSpec((_B, 1), lambda i: (i, 0)),
                  pl.BlockSpec((_B, 1), lambda i: (i, 0)),
                  pl.BlockSpec((_B, 1), lambda i: (i, 0)),
                  pl.BlockSpec((_B, 1), lambda i: (i, 0)),
                  pl.BlockSpec((_H1, _C), lambda i: (0, 0)),
                  pl.BlockSpec((1, _H1), lambda i: (0, 0))],
        out_specs=pl.BlockSpec((_B, _C), lambda i: (i, 0)),
        out_shape=jax.ShapeDtypeStruct((_N, _C), f32),
    )(p0, p1, y2, db0, db1, dc0, dc1, W2, b1)


def _tc_fin(p0, p1, y3, dc0, dc1, b2):
    def body(p0r, p1r, y3r, dc0r, dc1r, b2r, o_ref):
        dinvc = lax.rsqrt(dc0r[...] + dc1r[...] + 1.0)
        o_ref[...] = dinvc * (p0r[...] + p1r[...] + y3r[...]) + b2r[...]

    return pl.pallas_call(
        body,
        grid=(_G,),
        in_specs=[pl.BlockSpec((_B, _C), lambda i: (i, 0)),
                  pl.BlockSpec((_B, _C), lambda i: (i, 0)),
                  pl.BlockSpec((_B, _C), lambda i: (i, 0)),
                  pl.BlockSpec((_B, 1), lambda i: (i, 0)),
                  pl.BlockSpec((_B, 1), lambda i: (i, 0)),
                  pl.BlockSpec((1, _C), lambda i: (0, 0))],
        out_specs=pl.BlockSpec((_B, _C), lambda i: (i, 0)),
        out_shape=jax.ShapeDtypeStruct((_N, _C), f32),
    )(p0, p1, y3, dc0, dc1, b2)


def kernel(x, edge_index, edge_weight, tR_indices, p,
           W_ih, W_hh, b_ih, b_hh, W0, W1, b1, W2, b2):
    npad = _EPAD - _E
    pad_src = (jnp.arange(npad, dtype=i32) * 53) % _N
    pad_dst = _N + (jnp.arange(npad, dtype=i32) % (_NPAD - _N))
    src = jnp.concatenate([edge_index[0].astype(i32), pad_src])
    dst = jnp.concatenate([edge_index[1].astype(i32), pad_dst])
    trs = jnp.concatenate([tR_indices[0].astype(i32), pad_src])
    trd = jnp.concatenate([tR_indices[1].astype(i32), pad_dst])
    wp = jnp.concatenate([edge_weight, jnp.zeros((npad,), f32)])
    w_splat = jnp.broadcast_to(
        wp.reshape(_EPAD // 8, 8, 1), (_EPAD // 8, 8, 16)).reshape(_EPAD, 16)

    deg = _sc_deg(dst, trd, w_splat).reshape(_NC, 3, _NPAD, 16)
    da0 = deg[0, 0, :_N, 0:1]
    da1 = deg[1, 0, :_N, 0:1]
    db0 = deg[0, 1, :_N, 0:1]
    db1 = deg[1, 1, :_N, 0:1]
    dc0 = deg[0, 2, :_N, 0:1]
    dc1 = deg[1, 2, :_N, 0:1]

    score = _tc_score(x, p.reshape(_C, 1))[:, 0]
    vals, idx = lax.top_k(score, _C)
    x_tilde = x[idx] * vals[:, None]
    Wt = _tc_gru(x_tilde, W0, W_ih, W_hh,
                 b_ih.reshape(1, -1), b_hh.reshape(1, -1))

    y1 = _tc_y1(da0, da1, x, Wt)
    p1 = _spmm_w128(src, dst, w_splat, y1).reshape(_NC, _NPAD, _C)
    y2 = _tc_mid2(p1[0, :_N], p1[1, :_N], y1, da0, da1, db0, db1, W1)
    p2 = _spmm_16(src, dst, y2).reshape(_NC, _NPAD, _H1)
    y3 = _tc_mid3(p2[0, :_N], p2[1, :_N], y2, db0, db1, dc0, dc1,
                  W2, b1.reshape(1, -1))
    p3 = _spmm_128(trs, trd, y3).reshape(_NC, _NPAD, _C)
    return _tc_fin(p3[0, :_N], p3[1, :_N], y3, dc0, dc1, b2.reshape(1, -1))


# speedup vs baseline: 31.4742x; 1.2996x over previous
import functools

import jax
import jax.numpy as jnp
from jax import lax
from jax.experimental import pallas as pl
from jax.experimental.pallas import tpu as pltpu
from jax.experimental.pallas import tpu_sc as plsc

f32 = jnp.float32
i32 = jnp.int32

_N = 10000
_C = 128
_H1 = 16
_E = 320000
_NPAD = 10240
_CH = 128
_NC = 2
_NS = 16
_NW = _NC * _NS
_EPW = 10240
_EPAD = _EPW * _NW
_NCH = _EPW // _CH
_NQ = _NCH // 2
_RPT = _NPAD // _NS

_mesh = plsc.VectorSubcoreMesh(core_axis_name="c", subcore_axis_name="s")
_sc_params = pltpu.CompilerParams(use_tc_tiling_on_sc=False,
                                  needs_layout_passes=False)


def _zero_rows(buf, nrows, ncol16):
    def zb(i, _):
        for j in range(ncol16):
            buf[i, pl.ds(j * 16, 16)] = jnp.zeros((16,), f32)
        return _
    lax.fori_loop(0, nrows, zb, 0)


def _coff(base, t):
    return pl.multiple_of(base + t * _CH, _CH)


@functools.partial(
    pl.kernel,
    mesh=_mesh,
    compiler_params=_sc_params,
    out_type=jax.ShapeDtypeStruct((_NC * 3 * _NPAD, 16), f32),
    scratch_types=[
        pltpu.VMEM((_CH,), i32), pltpu.VMEM((_CH,), i32),
        pltpu.VMEM((_CH,), i32), pltpu.VMEM((_CH,), i32),
        pltpu.VMEM((_CH,), f32), pltpu.VMEM((_CH,), f32),
        pltpu.VMEM((_CH, 16), f32),
        pltpu.VMEM((_CH, 16), f32),
        pltpu.VMEM((64, 16), f32),
        pltpu.VMEM_SHARED((_NPAD, 16), f32),
        pltpu.VMEM_SHARED((_NPAD, 16), f32),
        pltpu.VMEM_SHARED((_NPAD, 16), f32),
        pltpu.SemaphoreType.DMA, pltpu.SemaphoreType.DMA,
    ],
)
def _sc_deg(dst_hbm, trd_hbm, w_hbm, out_hbm,
            dstA, dstB, trdA, trdB, wbA, wbB, wsc, obuf, zbuf,
            acc_a, acc_b, acc_c, semA, semB):
    cid = lax.axis_index("c")
    sid = lax.axis_index("s")
    _zero_rows(zbuf, 64, 1)
    ones16 = jnp.ones((16,), f32)

    def ob(i, carry):
        obuf[i, :] = ones16
        return carry

    lax.fori_loop(0, _CH, ob, 0)
    for j in range(_RPT // 64):
        r0 = sid * _RPT + j * 64
        pltpu.sync_copy(zbuf, acc_a.at[pl.ds(r0, 64), :])
        pltpu.sync_copy(zbuf, acc_b.at[pl.ds(r0, 64), :])
        pltpu.sync_copy(zbuf, acc_c.at[pl.ds(r0, 64), :])
    plsc.subcore_barrier()
    base = (sid * _NC + cid) * _EPW

    def loads(t, d_v, t_v, w_v, sem):
        off = _coff(base, t)
        pltpu.async_copy(dst_hbm.at[pl.ds(off, _CH)], d_v, sem)
        pltpu.async_copy(trd_hbm.at[pl.ds(off, _CH)], t_v, sem)
        pltpu.async_copy(w_hbm.at[pl.ds(off, _CH)], w_v, sem)

    def drain(t, d_v, t_v, w_v, sem):
        off = _coff(base, t)
        pltpu.make_async_copy(dst_hbm.at[pl.ds(off, _CH)], d_v, sem).wait()
        pltpu.make_async_copy(trd_hbm.at[pl.ds(off, _CH)], t_v, sem).wait()
        pltpu.make_async_copy(w_hbm.at[pl.ds(off, _CH)], w_v, sem).wait()

    def scatters(d_v, t_v, w_v):
        def build(i, carry):
            wsc[i, :] = plsc.load_gather(w_v, [jnp.full((16,), i, i32)])
            return carry
        lax.fori_loop(0, _CH, build, 0)
        pltpu.sync_copy(wsc, acc_a.at[d_v], add=True)
        pltpu.sync_copy(obuf, acc_b.at[d_v], add=True)
        pltpu.sync_copy(obuf, acc_c.at[t_v], add=True)

    loads(0, dstA, trdA, wbA, semA)
    loads(1, dstB, trdB, wbB, semB)

    def body(q, carry):
        t0 = 2 * q
        drain(t0, dstA, trdA, wbA, semA)
        scatters(dstA, trdA, wbA)

        @pl.when(q + 1 < _NQ)
        def _():
            loads(t0 + 2, dstA, trdA, wbA, semA)

        drain(t0 + 1, dstB, trdB, wbB, semB)
        scatters(dstB, trdB, wbB)

        @pl.when(q + 1 < _NQ)
        def _():
            loads(t0 + 3, dstB, trdB, wbB, semB)

        return carry

    lax.fori_loop(0, _NQ, body, 0)
    plsc.subcore_barrier()
    for j in range(_RPT // 64):
        r0 = sid * _RPT + j * 64
        for a, acc in enumerate((acc_a, acc_b, acc_c)):
            pltpu.sync_copy(acc.at[pl.ds(r0, 64), :],
                            out_hbm.at[pl.ds((cid * 3 + a) * _NPAD + r0, 64), :])


def _make_spmm(F, scale, small):
    scratch = [
        pltpu.VMEM((_CH,), i32), pltpu.VMEM((_CH,), i32),
        pltpu.VMEM((_CH,), i32), pltpu.VMEM((_CH,), i32),
        pltpu.VMEM((_CH, F), f32), pltpu.VMEM((_CH, F), f32),
        pltpu.VMEM((64, F), f32),
        pltpu.VMEM_SHARED((_NPAD, F), f32),
        pltpu.SemaphoreType.DMA, pltpu.SemaphoreType.DMA,
        pltpu.SemaphoreType.DMA, pltpu.SemaphoreType.DMA,
    ]
    if scale:
        scratch = [pltpu.VMEM((_CH,), f32),
                   pltpu.VMEM((_CH,), f32)] + scratch
    if small:
        scratch = [pltpu.VMEM((1000, F), f32),
                   pltpu.VMEM_SHARED((_N, F), f32)] + scratch

    @functools.partial(
        pl.kernel,
        mesh=_mesh,
        compiler_params=_sc_params,
        out_type=jax.ShapeDtypeStruct((_NC * _NPAD, F), f32),
        scratch_types=scratch,
    )
    def k(*args):
        src_hbm, dst_hbm = args[0], args[1]
        a = 2
        wsp_hbm = args[a] if scale else None
        a += 1 if scale else 0
        y_hbm, out_hbm = args[a], args[a + 1]
        a += 2
        if small:
            stage_v, y_spm = args[a], args[a + 1]
            a += 2
        if scale:
            wbA, wbB = args[a], args[a + 1]
            a += 2
        (srcA, srcB, dstA, dstB, rowsA, rowsB, zbuf, acc,
         semIA, semIB, semGA, semGB) = args[a:]

        cid = lax.axis_index("c")
        sid = lax.axis_index("s")
        _zero_rows(zbuf, 64, F // 16)
        for j in range(_RPT // 64):
            r0 = sid * _RPT + j * 64
            pltpu.sync_copy(zbuf, acc.at[pl.ds(r0, 64), :])
        if small:
            @pl.when(sid < _N // 1000)
            def _stage():
                y0 = pl.multiple_of(sid * 1000, 8)
                pltpu.sync_copy(y_hbm.at[pl.ds(y0, 1000), :], stage_v)
                pltpu.sync_copy(stage_v, y_spm.at[pl.ds(y0, 1000), :])
            ysrc = y_spm
        else:
            ysrc = y_hbm
        plsc.subcore_barrier()
        base = (sid * _NC + cid) * _EPW

        def loads(t, s_v, d_v, w_v, sem):
            off = _coff(base, t)
            pltpu.async_copy(src_hbm.at[pl.ds(off, _CH)], s_v, sem)
            pltpu.async_copy(dst_hbm.at[pl.ds(off, _CH)], d_v, sem)
            if scale:
                pltpu.async_copy(wsp_hbm.at[pl.ds(off, _CH)], w_v, sem)

        def drain_loads(t, s_v, d_v, w_v, sem):
            off = _coff(base, t)
            pltpu.make_async_copy(src_hbm.at[pl.ds(off, _CH)], s_v, sem).wait()
            pltpu.make_async_copy(dst_hbm.at[pl.ds(off, _CH)], d_v, sem).wait()
            if scale:
                pltpu.make_async_copy(
                    wsp_hbm.at[pl.ds(off, _CH)], w_v, sem).wait()

        def do_scale(rows_v, w_v):
            if scale:
                def srow(i, c2):
                    wspl = plsc.load_gather(w_v, [jnp.full((16,), i, i32)])
                    for j in range(F // 16):
                        rows_v[i, pl.ds(j * 16, 16)] = (
                            rows_v[i, pl.ds(j * 16, 16)] * wspl)
                    return c2
                lax.fori_loop(0, _CH, srow, 0)

        wbA_ = wbA if scale else None
        wbB_ = wbB if scale else None
        loads(0, srcA, dstA, wbA_, semIA)
        loads(1, srcB, dstB, wbB_, semIB)
        drain_loads(0, srcA, dstA, wbA_, semIA)
        pltpu.async_copy(ysrc.at[srcA], rowsA, semGA)

        def body(q, carry):
            t0 = 2 * q
            drain_loads(t0 + 1, srcB, dstB, wbB_, semIB)
            pltpu.async_copy(ysrc.at[srcB], rowsB, semGB)
            pltpu.make_async_copy(ysrc.at[srcA], rowsA, semGA).wait()
            do_scale(rowsA, wbA_)
            pltpu.sync_copy(rowsA, acc.at[dstA], add=True)

            @pl.when(q + 1 < _NQ)
            def _():
                loads(t0 + 2, srcA, dstA, wbA_, semIA)

            pltpu.make_async_copy(ysrc.at[srcB], rowsB, semGB).wait()
            do_scale(rowsB, wbB_)

            @pl.when(q + 1 < _NQ)
            def _():
                drain_loads(t0 + 2, srcA, dstA, wbA_, semIA)
                pltpu.async_copy(ysrc.at[srcA], rowsA, semGA)

            pltpu.sync_copy(rowsB, acc.at[dstB], add=True)

            @pl.when(q + 1 < _NQ)
            def _():
                loads(t0 + 3, srcB, dstB, wbB_, semIB)

            return carry

        lax.fori_loop(0, _NQ, body, 0)
        plsc.subcore_barrier()
        for j in range(_RPT // 64):
            r0 = sid * _RPT + j * 64
            pltpu.sync_copy(acc.at[pl.ds(r0, 64), :],
                            out_hbm.at[pl.ds(cid * _NPAD + r0, 64), :])

    return k


_spmm_w128 = _make_spmm(_C, True, False)
_spmm_128 = _make_spmm(_C, False, False)
_spmm_16 = _make_spmm(_H1, False, True)


_B = 2000
_G = _N // _B


def _tc_score(x, p2):
    def body(x_ref, p_ref, o_ref):
        pv = p_ref[...]
        nrm = jnp.sqrt(jnp.sum(pv * pv))
        o_ref[...] = jnp.tanh(
            jnp.dot(x_ref[...], pv, preferred_element_type=f32) / nrm)

    return pl.pallas_call(
        body,
        grid=(_G,),
        in_specs=[pl.BlockSpec((_B, _C), lambda i: (i, 0)),
                  pl.BlockSpec((_C, 1), lambda i: (0, 0))],
        out_specs=pl.BlockSpec((_B, 1), lambda i: (i, 0)),
        out_shape=jax.ShapeDtypeStruct((_N, 1), f32),
    )(x, p2)


def _tc_gru(xt, W0, Wih, Whh, bih, bhh):
    def body(xt_ref, w0_ref, wih_ref, whh_ref, bih_ref, bhh_ref, o_ref):
        cdims = (((1,), (1,)), ((), ()))
        gi = lax.dot_general(xt_ref[...], wih_ref[...], cdims,
                             preferred_element_type=f32) + bih_ref[...]
        gh = lax.dot_general(w0_ref[...], whh_ref[...], cdims,
                             preferred_element_type=f32) + bhh_ref[...]
        r = jax.nn.sigmoid(gi[:, :_C] + gh[:, :_C])
        z = jax.nn.sigmoid(gi[:, _C:2 * _C] + gh[:, _C:2 * _C])
        nn_ = jnp.tanh(gi[:, 2 * _C:] + r * gh[:, 2 * _C:])
        o_ref[...] = (1.0 - z) * nn_ + z * w0_ref[...]

    return pl.pallas_call(
        body,
        out_shape=jax.ShapeDtypeStruct((_C, _C), f32),
    )(xt, W0, Wih, Whh, bih, bhh)


def _tc_y1(da0, da1, x, W):
    def body(d0, d1, x_ref, w_ref, o_ref):
        dinv = lax.rsqrt(d0[...] + d1[...] + 1.0)
        o_ref[...] = dinv * jnp.dot(x_ref[...], w_ref[...],
                                    preferred_element_type=f32)

    return pl.pallas_call(
        body,
        grid=(_G,),
        in_specs=[pl.BlockSpec((_B, 1), lambda i: (i, 0)),
                  pl.BlockSpec((_B, 1), lambda i: (i, 0)),
                  pl.BlockSpec((_B, _C), lambda i: (i, 0)),
                  pl.BlockSpec((_C, _C), lambda i: (0, 0))],
        out_specs=pl.BlockSpec((_B, _C), lambda i: (i, 0)),
        out_shape=jax.ShapeDtypeStruct((_N, _C), f32),
    )(da0, da1, x, W)


def _tc_mid2(p0, p1, y1, da0, da1, db0, db1, W1):
    def body(p0r, p1r, y1r, da0r, da1r, db0r, db1r, w1r, o_ref):
        dinva = lax.rsqrt(da0r[...] + da1r[...] + 1.0)
        dinvb = lax.rsqrt(db0r[...] + db1r[...] + 1.0)
        h1 = jnp.maximum(dinva * (p0r[...] + p1r[...] + y1r[...]), 0.0)
        o_ref[...] = dinvb * jnp.dot(h1, w1r[...], preferred_element_type=f32)

    return pl.pallas_call(
        body,
        grid=(_G,),
        in_specs=[pl.BlockSpec((_B, _C), lambda i: (i, 0)),
                  pl.BlockSpec((_B, _C), lambda i: (i, 0)),
                  pl.BlockSpec((_B, _C), lambda i: (i, 0)),
                  pl.BlockSpec((_B, 1), lambda i: (i, 0)),
                  pl.BlockSpec((_B, 1), lambda i: (i, 0)),
                  pl.BlockSpec((_B, 1), lambda i: (i, 0)),
                  pl.BlockSpec((_B, 1), lambda i: (i, 0)),
                  pl.BlockSpec((_C, _H1), lambda i: (0, 0))],
        out_specs=pl.BlockSpec((_B, _H1), lambda i: (i, 0)),
        out_shape=jax.ShapeDtypeStruct((_N, _H1), f32),
    )(p0, p1, y1, da0, da1, db0, db1, W1)


def _tc_mid3(p0, p1, y2, db0, db1, dc0, dc1, W2, b1):
    def body(p0r, p1r, y2r, db0r, db1r, dc0r, dc1r, w2r, b1r, o_ref):
        dinvb = lax.rsqrt(db0r[...] + db1r[...] + 1.0)
        dinvc = lax.rsqrt(dc0r[...] + dc1r[...] + 1.0)
        h2 = jnp.maximum(
            dinvb * (p0r[...] + p1r[...] + y2r[...]) + b1r[...], 0.0)
        o_ref[...] = dinvc * jnp.dot(h2, w2r[...], preferred_element_type=f32)

    return pl.pallas_call(
        body,
        grid=(_G,),
        in_specs=[pl.BlockSpec((_B, _H1), lambda i: (i, 0)),
                  pl.BlockSpec((_B, _H1), lambda i: (i, 0)),
                  pl.BlockSpec((_B, _H1), lambda i: (i, 0)),
                  pl.BlockSpec((_B, 1), lambda i: (i, 0)),
                  pl.BlockSpec((_B, 1), lambda i: (i, 0)),
                  pl.BlockSpec((_B, 1), lambda i: (i, 0)),
                  pl.BlockSpec((_B, 1), lambda i: (i, 0)),
                  pl.BlockSpec((_H1, _C), lambda i: (0, 0)),
                  pl.BlockSpec((1, _H1), lambda i: (0, 0))],
        out_specs=pl.BlockSpec((_B, _C), lambda i: (i, 0)),
        out_shape=jax.ShapeDtypeStruct((_N, _C), f32),
    )(p0, p1, y2, db0, db1, dc0, dc1, W2, b1)


def _tc_fin(p0, p1, y3, dc0, dc1, b2):
    def body(p0r, p1r, y3r, dc0r, dc1r, b2r, o_ref):
        dinvc = lax.rsqrt(dc0r[...] + dc1r[...] + 1.0)
        o_ref[...] = dinvc * (p0r[...] + p1r[...] + y3r[...]) + b2r[...]

    return pl.pallas_call(
        body,
        grid=(_G,),
        in_specs=[pl.BlockSpec((_B, _C), lambda i: (i, 0)),
                  pl.BlockSpec((_B, _C), lambda i: (i, 0)),
                  pl.BlockSpec((_B, _C), lambda i: (i, 0)),
                  pl.BlockSpec((_B, 1), lambda i: (i, 0)),
                  pl.BlockSpec((_B, 1), lambda i: (i, 0)),
                  pl.BlockSpec((1, _C), lambda i: (0, 0))],
        out_specs=pl.BlockSpec((_B, _C), lambda i: (i, 0)),
        out_shape=jax.ShapeDtypeStruct((_N, _C), f32),
    )(p0, p1, y3, dc0, dc1, b2)


def kernel(x, edge_index, edge_weight, tR_indices, p,
           W_ih, W_hh, b_ih, b_hh, W0, W1, b1, W2, b2):
    npad = _EPAD - _E
    pad_src = (jnp.arange(npad, dtype=i32) * 53) % _N
    pad_dst = _N + (jnp.arange(npad, dtype=i32) % (_NPAD - _N))
    src = jnp.concatenate([edge_index[0].astype(i32), pad_src])
    dst = jnp.concatenate([edge_index[1].astype(i32), pad_dst])
    trs = jnp.concatenate([tR_indices[0].astype(i32), pad_src])
    trd = jnp.concatenate([tR_indices[1].astype(i32), pad_dst])
    wp = jnp.concatenate([edge_weight, jnp.zeros((npad,), f32)])

    deg = _sc_deg(dst, trd, wp).reshape(_NC, 3, _NPAD, 16)
    da0 = deg[0, 0, :_N, 0:1]
    da1 = deg[1, 0, :_N, 0:1]
    db0 = deg[0, 1, :_N, 0:1]
    db1 = deg[1, 1, :_N, 0:1]
    dc0 = deg[0, 2, :_N, 0:1]
    dc1 = deg[1, 2, :_N, 0:1]

    score = _tc_score(x, p.reshape(_C, 1))[:, 0]
    vals, idx = lax.top_k(score, _C)
    x_tilde = x[idx] * vals[:, None]
    Wt = _tc_gru(x_tilde, W0, W_ih, W_hh,
                 b_ih.reshape(1, -1), b_hh.reshape(1, -1))

    y1 = _tc_y1(da0, da1, x, Wt)
    p1 = _spmm_w128(src, dst, wp, y1).reshape(_NC, _NPAD, _C)
    y2 = _tc_mid2(p1[0, :_N], p1[1, :_N], y1, da0, da1, db0, db1, W1)
    p2 = _spmm_16(src, dst, y2).reshape(_NC, _NPAD, _H1)
    y3 = _tc_mid3(p2[0, :_N], p2[1, :_N], y2, db0, db1, dc0, dc1,
                  W2, b1.reshape(1, -1))
    p3 = _spmm_128(trs, trd, y3).reshape(_NC, _NPAD, _C)
    return _tc_fin(p3[0, :_N], p3[1, :_N], y3, dc0, dc1, b2.reshape(1, -1))


# speedup vs baseline: 37.1361x; 1.1799x over previous
import functools

import jax
import jax.numpy as jnp
from jax import lax
from jax.experimental import pallas as pl
from jax.experimental.pallas import tpu as pltpu
from jax.experimental.pallas import tpu_sc as plsc

f32 = jnp.float32
i32 = jnp.int32

_N = 10000
_C = 128
_H1 = 16
_E = 320000
_NPAD = 10240
_CH = 128
_NC = 2
_NS = 16
_NW = _NC * _NS
_EPW = 10240
_EPAD = _EPW * _NW
_NCH = _EPW // _CH
_NQ = _NCH // 2
_RPT = _NPAD // _NS

_mesh = plsc.VectorSubcoreMesh(core_axis_name="c", subcore_axis_name="s")
_sc_params = pltpu.CompilerParams(use_tc_tiling_on_sc=False,
                                  needs_layout_passes=False)


def _zero_rows(buf, nrows, ncol16):
    def zb(i, _):
        for j in range(ncol16):
            buf[i, pl.ds(j * 16, 16)] = jnp.zeros((16,), f32)
        return _
    lax.fori_loop(0, nrows, zb, 0)


def _coff(base, t):
    return pl.multiple_of(base + t * _CH, _CH)


@functools.partial(
    pl.kernel,
    mesh=_mesh,
    compiler_params=_sc_params,
    out_type=jax.ShapeDtypeStruct((_NC * 3 * _NPAD, 16), f32),
    scratch_types=[
        pltpu.VMEM((_CH,), i32), pltpu.VMEM((_CH,), i32),
        pltpu.VMEM((_CH,), i32), pltpu.VMEM((_CH,), i32),
        pltpu.VMEM((_CH,), i32), pltpu.VMEM((_CH,), i32),
        pltpu.VMEM((_CH,), f32), pltpu.VMEM((_CH,), f32),
        pltpu.VMEM((_CH, 16), f32), pltpu.VMEM((_CH, 16), f32),
        pltpu.VMEM((_CH, 16), f32),
        pltpu.VMEM((64, 16), f32),
        pltpu.VMEM_SHARED((_NPAD, 16), f32),
        pltpu.VMEM_SHARED((_NPAD, 16), f32),
        pltpu.VMEM_SHARED((_NPAD, 16), f32),
        pltpu.SemaphoreType.DMA, pltpu.SemaphoreType.DMA,
        pltpu.SemaphoreType.DMA, pltpu.SemaphoreType.DMA,
    ],
)
def _sc_deg(ei_hbm, tr_hbm, w_hbm, out_hbm,
            dstA, dstB, trdA, trdB, dstS, trdS, wbA, wbB, wscA, wscB, obuf,
            zbuf, acc_a, acc_b, acc_c, semA, semB, semSA, semSB):
    cid = lax.axis_index("c")
    sid = lax.axis_index("s")
    _zero_rows(zbuf, 64, 1)
    ones16 = jnp.ones((16,), f32)

    def ob(i, carry):
        obuf[i, :] = ones16
        return carry

    lax.fori_loop(0, _CH, ob, 0)
    for j in range(_RPT // 64):
        r0 = sid * _RPT + j * 64
        pltpu.sync_copy(zbuf, acc_a.at[pl.ds(r0, 64), :])
        pltpu.sync_copy(zbuf, acc_b.at[pl.ds(r0, 64), :])
        pltpu.sync_copy(zbuf, acc_c.at[pl.ds(r0, 64), :])
    plsc.subcore_barrier()
    base = (sid * _NC + cid) * _EPW

    def loads(t, d_v, t_v, w_v, sem):
        off = _coff(base, t)
        pltpu.async_copy(ei_hbm.at[1, pl.ds(off, _CH)], d_v, sem)
        pltpu.async_copy(tr_hbm.at[1, pl.ds(off, _CH)], t_v, sem)
        pltpu.async_copy(w_hbm.at[pl.ds(off, _CH)], w_v, sem)

    def drain(t, d_v, t_v, w_v, sem):
        off = _coff(base, t)
        pltpu.make_async_copy(ei_hbm.at[1, pl.ds(off, _CH)], d_v, sem).wait()
        pltpu.make_async_copy(tr_hbm.at[1, pl.ds(off, _CH)], t_v, sem).wait()
        pltpu.make_async_copy(w_hbm.at[pl.ds(off, _CH)], w_v, sem).wait()

    def build_wsc(wsc, w_v):
        def build(i, carry):
            for u in range(4):
                k = i * 4 + u
                wsc[k, :] = plsc.load_gather(w_v, [jnp.full((16,), k, i32)])
            return carry
        lax.fori_loop(0, _CH // 4, build, 0)

    def copy_idx(dst8, src8):
        for g in range(_CH // 16):
            dst8[pl.ds(g * 16, 16)] = src8[pl.ds(g * 16, 16)]

    def scatters(d_v, t_v, wsc, semS):
        copy_idx(dstS, d_v)
        copy_idx(trdS, t_v)
        pltpu.async_copy(wsc, acc_a.at[dstS], semS, add=True)
        pltpu.async_copy(obuf, acc_b.at[dstS], semS, add=True)
        pltpu.async_copy(obuf, acc_c.at[trdS], semS, add=True)

    def wait_scat(semS):
        pltpu.make_async_copy(wscA, acc_a.at[dstS], semS).wait()
        pltpu.make_async_copy(obuf, acc_b.at[dstS], semS).wait()
        pltpu.make_async_copy(obuf, acc_c.at[trdS], semS).wait()

    loads(0, dstA, trdA, wbA, semA)
    loads(1, dstB, trdB, wbB, semB)

    def body(q, carry):
        t0 = 2 * q
        drain(t0, dstA, trdA, wbA, semA)
        build_wsc(wscA, wbA)

        @pl.when(q > 0)
        def _():
            wait_scat(semSB)

        scatters(dstA, trdA, wscA, semSA)

        @pl.when(q + 1 < _NQ)
        def _():
            loads(t0 + 2, dstA, trdA, wbA, semA)

        drain(t0 + 1, dstB, trdB, wbB, semB)
        build_wsc(wscB, wbB)
        wait_scat(semSA)
        scatters(dstB, trdB, wscB, semSB)

        @pl.when(q + 1 < _NQ)
        def _():
            loads(t0 + 3, dstB, trdB, wbB, semB)

        return carry

    lax.fori_loop(0, _NQ, body, 0)
    wait_scat(semSB)
    plsc.subcore_barrier()
    for j in range(_RPT // 64):
        r0 = sid * _RPT + j * 64
        for a, acc in enumerate((acc_a, acc_b, acc_c)):
            pltpu.sync_copy(acc.at[pl.ds(r0, 64), :],
                            out_hbm.at[pl.ds((cid * 3 + a) * _NPAD + r0, 64), :])


def _make_spmm(F, scale, small):
    scratch = [
        pltpu.VMEM((_CH,), i32), pltpu.VMEM((_CH,), i32),
        pltpu.VMEM((_CH,), i32), pltpu.VMEM((_CH,), i32),
        pltpu.VMEM((_CH,), i32), pltpu.VMEM((_CH,), i32),
        pltpu.VMEM((_CH, F), f32), pltpu.VMEM((_CH, F), f32),
        pltpu.VMEM((64, F), f32),
        pltpu.VMEM_SHARED((_NPAD, F), f32),
        pltpu.SemaphoreType.DMA, pltpu.SemaphoreType.DMA,
        pltpu.SemaphoreType.DMA, pltpu.SemaphoreType.DMA,
        pltpu.SemaphoreType.DMA, pltpu.SemaphoreType.DMA,
    ]
    if scale:
        scratch = [pltpu.VMEM((_CH,), f32),
                   pltpu.VMEM((_CH,), f32)] + scratch
    if small:
        scratch = [pltpu.VMEM((1000, F), f32),
                   pltpu.VMEM_SHARED((_N, F), f32)] + scratch

    @functools.partial(
        pl.kernel,
        mesh=_mesh,
        compiler_params=_sc_params,
        out_type=jax.ShapeDtypeStruct((_NC * _NPAD, F), f32),
        scratch_types=scratch,
    )
    def k(*args):
        ei_hbm = args[0]
        a = 1
        wsp_hbm = args[a] if scale else None
        a += 1 if scale else 0
        y_hbm, out_hbm = args[a], args[a + 1]
        a += 2
        if small:
            stage_v, y_spm = args[a], args[a + 1]
            a += 2
        if scale:
            wbA, wbB = args[a], args[a + 1]
            a += 2
        (srcA, srcB, dstA, dstB, dstSA, dstSB, rowsA, rowsB, zbuf, acc,
         semIA, semIB, semGA, semGB, semSA, semSB) = args[a:]

        cid = lax.axis_index("c")
        sid = lax.axis_index("s")
        _zero_rows(zbuf, 64, F // 16)
        for j in range(_RPT // 64):
            r0 = sid * _RPT + j * 64
            pltpu.sync_copy(zbuf, acc.at[pl.ds(r0, 64), :])
        if small:
            @pl.when(sid < _N // 1000)
            def _stage():
                y0 = pl.multiple_of(sid * 1000, 8)
                pltpu.sync_copy(y_hbm.at[pl.ds(y0, 1000), :], stage_v)
                pltpu.sync_copy(stage_v, y_spm.at[pl.ds(y0, 1000), :])
            ysrc = y_spm
        else:
            ysrc = y_hbm
        plsc.subcore_barrier()
        base = (sid * _NC + cid) * _EPW

        def loads(t, s_v, d_v, w_v, sem):
            off = _coff(base, t)
            pltpu.async_copy(ei_hbm.at[0, pl.ds(off, _CH)], s_v, sem)
            pltpu.async_copy(ei_hbm.at[1, pl.ds(off, _CH)], d_v, sem)
            if scale:
                pltpu.async_copy(wsp_hbm.at[pl.ds(off, _CH)], w_v, sem)

        def drain_loads(t, s_v, d_v, w_v, sem):
            off = _coff(base, t)
            pltpu.make_async_copy(ei_hbm.at[0, pl.ds(off, _CH)], s_v, sem).wait()
            pltpu.make_async_copy(ei_hbm.at[1, pl.ds(off, _CH)], d_v, sem).wait()
            if scale:
                pltpu.make_async_copy(
                    wsp_hbm.at[pl.ds(off, _CH)], w_v, sem).wait()

        def do_scale(rows_v, w_v):
            if scale:
                def srow(i, c2):
                    for u in range(4):
                        kk = i * 4 + u
                        wspl = plsc.load_gather(
                            w_v, [jnp.full((16,), kk, i32)])
                        for j in range(F // 16):
                            rows_v[kk, pl.ds(j * 16, 16)] = (
                                rows_v[kk, pl.ds(j * 16, 16)] * wspl)
                    return c2
                lax.fori_loop(0, _CH // 4, srow, 0)

        def copy_idx(d8, s8):
            for g in range(_CH // 16):
                d8[pl.ds(g * 16, 16)] = s8[pl.ds(g * 16, 16)]

        wbA_ = wbA if scale else None
        wbB_ = wbB if scale else None
        loads(0, srcA, dstA, wbA_, semIA)
        loads(1, srcB, dstB, wbB_, semIB)
        drain_loads(0, srcA, dstA, wbA_, semIA)
        pltpu.async_copy(ysrc.at[srcA], rowsA, semGA)

        def body(q, carry):
            t0 = 2 * q
            drain_loads(t0 + 1, srcB, dstB, wbB_, semIB)

            @pl.when(q > 0)
            def _():
                pltpu.make_async_copy(rowsB, acc.at[dstSB], semSB).wait()

            pltpu.async_copy(ysrc.at[srcB], rowsB, semGB)
            pltpu.make_async_copy(ysrc.at[srcA], rowsA, semGA).wait()
            do_scale(rowsA, wbA_)
            copy_idx(dstSA, dstA)
            pltpu.async_copy(rowsA, acc.at[dstSA], semSA, add=True)

            @pl.when(q + 1 < _NQ)
            def _():
                loads(t0 + 2, srcA, dstA, wbA_, semIA)

            pltpu.make_async_copy(ysrc.at[srcB], rowsB, semGB).wait()
            do_scale(rowsB, wbB_)
            copy_idx(dstSB, dstB)
            pltpu.async_copy(rowsB, acc.at[dstSB], semSB, add=True)

            @pl.when(q + 1 < _NQ)
            def _():
                drain_loads(t0 + 2, srcA, dstA, wbA_, semIA)
                pltpu.make_async_copy(rowsA, acc.at[dstSA], semSA).wait()
                pltpu.async_copy(ysrc.at[srcA], rowsA, semGA)
                loads(t0 + 3, srcB, dstB, wbB_, semIB)

            return carry

        lax.fori_loop(0, _NQ, body, 0)
        pltpu.make_async_copy(rowsA, acc.at[dstSA], semSA).wait()
        pltpu.make_async_copy(rowsB, acc.at[dstSB], semSB).wait()
        plsc.subcore_barrier()
        for j in range(_RPT // 64):
            r0 = sid * _RPT + j * 64
            pltpu.sync_copy(acc.at[pl.ds(r0, 64), :],
                            out_hbm.at[pl.ds(cid * _NPAD + r0, 64), :])

    return k


_spmm_w128 = _make_spmm(_C, True, False)
_spmm_128 = _make_spmm(_C, False, False)
_spmm_16 = _make_spmm(_H1, False, True)


_B = 2000
_G = _N // _B


def _tc_score(x, p2):
    def body(x_ref, p_ref, o_ref):
        pv = p_ref[...]
        nrm = jnp.sqrt(jnp.sum(pv * pv))
        o_ref[...] = jnp.tanh(
            jnp.dot(x_ref[...], pv, preferred_element_type=f32) / nrm)

    return pl.pallas_call(
        body,
        grid=(_G,),
        in_specs=[pl.BlockSpec((_B, _C), lambda i: (i, 0)),
                  pl.BlockSpec((_C, 1), lambda i: (0, 0))],
        out_specs=pl.BlockSpec((_B, 1), lambda i: (i, 0)),
        out_shape=jax.ShapeDtypeStruct((_N, 1), f32),
    )(x, p2)


def _tc_gru(xt, W0, Wih, Whh, bih, bhh):
    def body(xt_ref, w0_ref, wih_ref, whh_ref, bih_ref, bhh_ref, o_ref):
        cdims = (((1,), (1,)), ((), ()))
        gi = lax.dot_general(xt_ref[...], wih_ref[...], cdims,
                             preferred_element_type=f32) + bih_ref[...]
        gh = lax.dot_general(w0_ref[...], whh_ref[...], cdims,
                             preferred_element_type=f32) + bhh_ref[...]
        r = jax.nn.sigmoid(gi[:, :_C] + gh[:, :_C])
        z = jax.nn.sigmoid(gi[:, _C:2 * _C] + gh[:, _C:2 * _C])
        nn_ = jnp.tanh(gi[:, 2 * _C:] + r * gh[:, 2 * _C:])
        o_ref[...] = (1.0 - z) * nn_ + z * w0_ref[...]

    return pl.pallas_call(
        body,
        out_shape=jax.ShapeDtypeStruct((_C, _C), f32),
    )(xt, W0, Wih, Whh, bih, bhh)


def _tc_y1(da0, da1, x, W):
    def body(d0, d1, x_ref, w_ref, o_ref):
        dinv = lax.rsqrt(d0[...] + d1[...] + 1.0)
        o_ref[...] = dinv * jnp.dot(x_ref[...], w_ref[...],
                                    preferred_element_type=f32)

    return pl.pallas_call(
        body,
        grid=(_G,),
        in_specs=[pl.BlockSpec((_B, 1), lambda i: (i, 0)),
                  pl.BlockSpec((_B, 1), lambda i: (i, 0)),
                  pl.BlockSpec((_B, _C), lambda i: (i, 0)),
                  pl.BlockSpec((_C, _C), lambda i: (0, 0))],
        out_specs=pl.BlockSpec((_B, _C), lambda i: (i, 0)),
        out_shape=jax.ShapeDtypeStruct((_N, _C), f32),
    )(da0, da1, x, W)


def _tc_mid2(p0, p1, y1, da0, da1, db0, db1, W1):
    def body(p0r, p1r, y1r, da0r, da1r, db0r, db1r, w1r, o_ref):
        dinva = lax.rsqrt(da0r[...] + da1r[...] + 1.0)
        dinvb = lax.rsqrt(db0r[...] + db1r[...] + 1.0)
        h1 = jnp.maximum(dinva * (p0r[...] + p1r[...] + y1r[...]), 0.0)
        o_ref[...] = dinvb * jnp.dot(h1, w1r[...], preferred_element_type=f32)

    return pl.pallas_call(
        body,
        grid=(_G,),
        in_specs=[pl.BlockSpec((_B, _C), lambda i: (i, 0)),
                  pl.BlockSpec((_B, _C), lambda i: (i, 0)),
                  pl.BlockSpec((_B, _C), lambda i: (i, 0)),
                  pl.BlockSpec((_B, 1), lambda i: (i, 0)),
                  pl.BlockSpec((_B, 1), lambda i: (i, 0)),
                  pl.BlockSpec((_B, 1), lambda i: (i, 0)),
                  pl.BlockSpec((_B, 1), lambda i: (i, 0)),
                  pl.BlockSpec((_C, _H1), lambda i: (0, 0))],
        out_specs=pl.BlockSpec((_B, _H1), lambda i: (i, 0)),
        out_shape=jax.ShapeDtypeStruct((_N, _H1), f32),
    )(p0, p1, y1, da0, da1, db0, db1, W1)


def _tc_mid3(p0, p1, y2, db0, db1, dc0, dc1, W2, b1):
    def body(p0r, p1r, y2r, db0r, db1r, dc0r, dc1r, w2r, b1r, o_ref):
        dinvb = lax.rsqrt(db0r[...] + db1r[...] + 1.0)
        dinvc = lax.rsqrt(dc0r[...] + dc1r[...] + 1.0)
        h2 = jnp.maximum(
            dinvb * (p0r[...] + p1r[...] + y2r[...]) + b1r[...], 0.0)
        o_ref[...] = dinvc * jnp.dot(h2, w2r[...], preferred_element_type=f32)

    return pl.pallas_call(
        body,
        grid=(_G,),
        in_specs=[pl.BlockSpec((_B, _H1), lambda i: (i, 0)),
                  pl.BlockSpec((_B, _H1), lambda i: (i, 0)),
                  pl.BlockSpec((_B, _H1), lambda i: (i, 0)),
                  pl.BlockSpec((_B, 1), lambda i: (i, 0)),
                  pl.BlockSpec((_B, 1), lambda i: (i, 0)),
                  pl.BlockSpec((_B, 1), lambda i: (i, 0)),
                  pl.BlockSpec((_B, 1), lambda i: (i, 0)),
                  pl.BlockSpec((_H1, _C), lambda i: (0, 0)),
                  pl.BlockSpec((1, _H1), lambda i: (0, 0))],
        out_specs=pl.BlockSpec((_B, _C), lambda i: (i, 0)),
        out_shape=jax.ShapeDtypeStruct((_N, _C), f32),
    )(p0, p1, y2, db0, db1, dc0, dc1, W2, b1)


def _tc_fin(p0, p1, y3, dc0, dc1, b2):
    def body(p0r, p1r, y3r, dc0r, dc1r, b2r, o_ref):
        dinvc = lax.rsqrt(dc0r[...] + dc1r[...] + 1.0)
        o_ref[...] = dinvc * (p0r[...] + p1r[...] + y3r[...]) + b2r[...]

    return pl.pallas_call(
        body,
        grid=(_G,),
        in_specs=[pl.BlockSpec((_B, _C), lambda i: (i, 0)),
                  pl.BlockSpec((_B, _C), lambda i: (i, 0)),
                  pl.BlockSpec((_B, _C), lambda i: (i, 0)),
                  pl.BlockSpec((_B, 1), lambda i: (i, 0)),
                  pl.BlockSpec((_B, 1), lambda i: (i, 0)),
                  pl.BlockSpec((1, _C), lambda i: (0, 0))],
        out_specs=pl.BlockSpec((_B, _C), lambda i: (i, 0)),
        out_shape=jax.ShapeDtypeStruct((_N, _C), f32),
    )(p0, p1, y3, dc0, dc1, b2)


def kernel(x, edge_index, edge_weight, tR_indices, p,
           W_ih, W_hh, b_ih, b_hh, W0, W1, b1, W2, b2):
    npad = _EPAD - _E
    pad_src = (jnp.arange(npad, dtype=i32) * 53) % _N
    pad_dst = _N + (jnp.arange(npad, dtype=i32) % (_NPAD - _N))
    pad_blk = jnp.stack([pad_src, pad_dst])
    eip = jnp.concatenate([edge_index.astype(i32), pad_blk], axis=1)
    trp = jnp.concatenate([tR_indices.astype(i32), pad_blk], axis=1)
    wp = jnp.concatenate([edge_weight, jnp.zeros((npad,), f32)])

    deg = _sc_deg(eip, trp, wp).reshape(_NC, 3, _NPAD, 16)
    da0 = deg[0, 0, :_N, 0:1]
    da1 = deg[1, 0, :_N, 0:1]
    db0 = deg[0, 1, :_N, 0:1]
    db1 = deg[1, 1, :_N, 0:1]
    dc0 = deg[0, 2, :_N, 0:1]
    dc1 = deg[1, 2, :_N, 0:1]

    score = _tc_score(x, p.reshape(_C, 1))[:, 0]
    vals, idx = lax.top_k(score, _C)
    x_tilde = x[idx] * vals[:, None]
    Wt = _tc_gru(x_tilde, W0, W_ih, W_hh,
                 b_ih.reshape(1, -1), b_hh.reshape(1, -1))

    y1 = _tc_y1(da0, da1, x, Wt)
    p1 = _spmm_w128(eip, wp, y1).reshape(_NC, _NPAD, _C)
    y2 = _tc_mid2(p1[0, :_N], p1[1, :_N], y1, da0, da1, db0, db1, W1)
    p2 = _spmm_16(eip, y2).reshape(_NC, _NPAD, _H1)
    y3 = _tc_mid3(p2[0, :_N], p2[1, :_N], y2, db0, db1, dc0, dc1,
                  W2, b1.reshape(1, -1))
    p3 = _spmm_128(trp, y3).reshape(_NC, _NPAD, _C)
    return _tc_fin(p3[0, :_N], p3[1, :_N], y3, dc0, dc1, b2.reshape(1, -1))


# speedup vs baseline: 38.6180x; 1.0399x over previous
import functools

import jax
import jax.numpy as jnp
from jax import lax
from jax.experimental import pallas as pl
from jax.experimental.pallas import tpu as pltpu
from jax.experimental.pallas import tpu_sc as plsc

f32 = jnp.float32
i32 = jnp.int32

_N = 10000
_C = 128
_H1 = 16
_E = 320000
_NPAD = 10240
_CH = 128
_NC = 2
_NS = 16
_NW = _NC * _NS
_EPW = 10240
_EPAD = _EPW * _NW
_NCH = _EPW // _CH
_NQ = _NCH // 2
_RPT = _NPAD // _NS

_mesh = plsc.VectorSubcoreMesh(core_axis_name="c", subcore_axis_name="s")
_sc_params = pltpu.CompilerParams(use_tc_tiling_on_sc=False,
                                  needs_layout_passes=False)


def _zero_rows(buf, nrows, ncol16):
    def zb(i, _):
        for j in range(ncol16):
            buf[i, pl.ds(j * 16, 16)] = jnp.zeros((16,), f32)
        return _
    lax.fori_loop(0, nrows, zb, 0)


def _coff(base, t):
    return pl.multiple_of(base + t * _CH, _CH)


@functools.partial(
    pl.kernel,
    mesh=_mesh,
    compiler_params=_sc_params,
    out_type=jax.ShapeDtypeStruct((_NC * 3 * _NPAD, 16), f32),
    scratch_types=[
        pltpu.VMEM((_CH,), i32), pltpu.VMEM((_CH,), i32),
        pltpu.VMEM((_CH,), i32), pltpu.VMEM((_CH,), i32),
        pltpu.VMEM((_CH,), i32), pltpu.VMEM((_CH,), i32),
        pltpu.VMEM((_CH,), f32), pltpu.VMEM((_CH,), f32),
        pltpu.VMEM((_CH, 16), f32), pltpu.VMEM((_CH, 16), f32),
        pltpu.VMEM((_CH, 16), f32),
        pltpu.VMEM((64, 16), f32),
        pltpu.VMEM_SHARED((_NPAD, 16), f32),
        pltpu.VMEM_SHARED((_NPAD, 16), f32),
        pltpu.VMEM_SHARED((_NPAD, 16), f32),
        pltpu.SemaphoreType.DMA, pltpu.SemaphoreType.DMA,
        pltpu.SemaphoreType.DMA, pltpu.SemaphoreType.DMA,
    ],
)
def _sc_deg(ei_hbm, tr_hbm, w_hbm, out_hbm,
            dstA, dstB, trdA, trdB, dstS, trdS, wbA, wbB, wscA, wscB, obuf,
            zbuf, acc_a, acc_b, acc_c, semA, semB, semSA, semSB):
    cid = lax.axis_index("c")
    sid = lax.axis_index("s")
    _zero_rows(zbuf, 64, 1)
    ones16 = jnp.ones((16,), f32)

    def ob(i, carry):
        obuf[i, :] = ones16
        return carry

    lax.fori_loop(0, _CH, ob, 0)
    for j in range(_RPT // 64):
        r0 = sid * _RPT + j * 64
        pltpu.sync_copy(zbuf, acc_a.at[pl.ds(r0, 64), :])
        pltpu.sync_copy(zbuf, acc_b.at[pl.ds(r0, 64), :])
        pltpu.sync_copy(zbuf, acc_c.at[pl.ds(r0, 64), :])
    plsc.subcore_barrier()
    base = (sid * _NC + cid) * _EPW

    def loads(t, d_v, t_v, w_v, sem):
        off = _coff(base, t)
        pltpu.async_copy(ei_hbm.at[1, pl.ds(off, _CH)], d_v, sem)
        pltpu.async_copy(tr_hbm.at[1, pl.ds(off, _CH)], t_v, sem)
        pltpu.async_copy(w_hbm.at[pl.ds(off, _CH)], w_v, sem)

    def drain(t, d_v, t_v, w_v, sem):
        off = _coff(base, t)
        pltpu.make_async_copy(ei_hbm.at[1, pl.ds(off, _CH)], d_v, sem).wait()
        pltpu.make_async_copy(tr_hbm.at[1, pl.ds(off, _CH)], t_v, sem).wait()
        pltpu.make_async_copy(w_hbm.at[pl.ds(off, _CH)], w_v, sem).wait()

    def build_wsc(wsc, w_v):
        def build(i, carry):
            for u in range(4):
                k = i * 4 + u
                wsc[k, :] = plsc.load_gather(w_v, [jnp.full((16,), k, i32)])
            return carry
        lax.fori_loop(0, _CH // 4, build, 0)

    def copy_idx(dst8, src8):
        for g in range(_CH // 16):
            dst8[pl.ds(g * 16, 16)] = src8[pl.ds(g * 16, 16)]

    def scatters(d_v, t_v, wsc, semS):
        copy_idx(dstS, d_v)
        copy_idx(trdS, t_v)
        pltpu.async_copy(wsc, acc_a.at[dstS], semS, add=True)
        pltpu.async_copy(obuf, acc_b.at[dstS], semS, add=True)
        pltpu.async_copy(obuf, acc_c.at[trdS], semS, add=True)

    def wait_scat(semS):
        pltpu.make_async_copy(wscA, acc_a.at[dstS], semS).wait()
        pltpu.make_async_copy(obuf, acc_b.at[dstS], semS).wait()
        pltpu.make_async_copy(obuf, acc_c.at[trdS], semS).wait()

    loads(0, dstA, trdA, wbA, semA)
    loads(1, dstB, trdB, wbB, semB)

    def body(q, carry):
        t0 = 2 * q
        drain(t0, dstA, trdA, wbA, semA)
        build_wsc(wscA, wbA)

        @pl.when(q > 0)
        def _():
            wait_scat(semSB)

        scatters(dstA, trdA, wscA, semSA)

        @pl.when(q + 1 < _NQ)
        def _():
            loads(t0 + 2, dstA, trdA, wbA, semA)

        drain(t0 + 1, dstB, trdB, wbB, semB)
        build_wsc(wscB, wbB)
        wait_scat(semSA)
        scatters(dstB, trdB, wscB, semSB)

        @pl.when(q + 1 < _NQ)
        def _():
            loads(t0 + 3, dstB, trdB, wbB, semB)

        return carry

    lax.fori_loop(0, _NQ, body, 0)
    wait_scat(semSB)
    plsc.subcore_barrier()
    for j in range(_RPT // 64):
        r0 = sid * _RPT + j * 64
        for a, acc in enumerate((acc_a, acc_b, acc_c)):
            pltpu.sync_copy(acc.at[pl.ds(r0, 64), :],
                            out_hbm.at[pl.ds((cid * 3 + a) * _NPAD + r0, 64), :])


def _make_spmm(F, scale, small):
    scratch = [
        pltpu.VMEM((_CH,), i32), pltpu.VMEM((_CH,), i32),
        pltpu.VMEM((_CH,), i32), pltpu.VMEM((_CH,), i32),
        pltpu.VMEM((_CH,), i32), pltpu.VMEM((_CH,), i32),
        pltpu.VMEM((_CH, F), f32), pltpu.VMEM((_CH, F), f32),
        pltpu.VMEM((64, F), f32),
        pltpu.VMEM_SHARED((_NPAD, F), f32),
        pltpu.SemaphoreType.DMA, pltpu.SemaphoreType.DMA,
        pltpu.SemaphoreType.DMA, pltpu.SemaphoreType.DMA,
        pltpu.SemaphoreType.DMA, pltpu.SemaphoreType.DMA,
    ]
    if scale:
        scratch = [pltpu.VMEM((_CH,), f32),
                   pltpu.VMEM((_CH,), f32)] + scratch
    if small:
        scratch = [pltpu.VMEM((1000, F), f32),
                   pltpu.VMEM_SHARED((_N, F), f32)] + scratch

    @functools.partial(
        pl.kernel,
        mesh=_mesh,
        compiler_params=_sc_params,
        out_type=jax.ShapeDtypeStruct((_NC * _NPAD, F), f32),
        scratch_types=scratch,
    )
    def k(*args):
        ei_hbm = args[0]
        a = 1
        wsp_hbm = args[a] if scale else None
        a += 1 if scale else 0
        y_hbm, out_hbm = args[a], args[a + 1]
        a += 2
        if small:
            stage_v, y_spm = args[a], args[a + 1]
            a += 2
        if scale:
            wbA, wbB = args[a], args[a + 1]
            a += 2
        (srcA, srcB, dstA, dstB, dstSA, dstSB, rowsA, rowsB, zbuf, acc,
         semIA, semIB, semGA, semGB, semSA, semSB) = args[a:]

        cid = lax.axis_index("c")
        sid = lax.axis_index("s")
        _zero_rows(zbuf, 64, F // 16)
        for j in range(_RPT // 64):
            r0 = sid * _RPT + j * 64
            pltpu.sync_copy(zbuf, acc.at[pl.ds(r0, 64), :])
        if small:
            @pl.when(sid < _N // 1000)
            def _stage():
                y0 = pl.multiple_of(sid * 1000, 8)
                pltpu.sync_copy(y_hbm.at[pl.ds(y0, 1000), :], stage_v)
                pltpu.sync_copy(stage_v, y_spm.at[pl.ds(y0, 1000), :])
            ysrc = y_spm
        else:
            ysrc = y_hbm
        plsc.subcore_barrier()
        base = (sid * _NC + cid) * _EPW

        def loads(t, s_v, d_v, w_v, sem):
            off = _coff(base, t)
            pltpu.async_copy(ei_hbm.at[0, pl.ds(off, _CH)], s_v, sem)
            pltpu.async_copy(ei_hbm.at[1, pl.ds(off, _CH)], d_v, sem)
            if scale:
                pltpu.async_copy(wsp_hbm.at[pl.ds(off, _CH)], w_v, sem)

        def drain_loads(t, s_v, d_v, w_v, sem):
            off = _coff(base, t)
            pltpu.make_async_copy(ei_hbm.at[0, pl.ds(off, _CH)], s_v, sem).wait()
            pltpu.make_async_copy(ei_hbm.at[1, pl.ds(off, _CH)], d_v, sem).wait()
            if scale:
                pltpu.make_async_copy(
                    wsp_hbm.at[pl.ds(off, _CH)], w_v, sem).wait()

        def do_scale(rows_v, w_v):
            if scale:
                def srow(i, c2):
                    for u in range(4):
                        kk = i * 4 + u
                        wspl = plsc.load_gather(
                            w_v, [jnp.full((16,), kk, i32)])
                        for j in range(F // 16):
                            rows_v[kk, pl.ds(j * 16, 16)] = (
                                rows_v[kk, pl.ds(j * 16, 16)] * wspl)
                    return c2
                lax.fori_loop(0, _CH // 4, srow, 0)

        def copy_idx(d8, s8):
            for g in range(_CH // 16):
                d8[pl.ds(g * 16, 16)] = s8[pl.ds(g * 16, 16)]

        wbA_ = wbA if scale else None
        wbB_ = wbB if scale else None
        loads(0, srcA, dstA, wbA_, semIA)
        loads(1, srcB, dstB, wbB_, semIB)
        drain_loads(0, srcA, dstA, wbA_, semIA)
        pltpu.async_copy(ysrc.at[srcA], rowsA, semGA)

        def body(q, carry):
            t0 = 2 * q
            drain_loads(t0 + 1, srcB, dstB, wbB_, semIB)

            @pl.when(q > 0)
            def _():
                pltpu.make_async_copy(rowsB, acc.at[dstSB], semSB).wait()

            pltpu.async_copy(ysrc.at[srcB], rowsB, semGB)
            pltpu.make_async_copy(ysrc.at[srcA], rowsA, semGA).wait()
            do_scale(rowsA, wbA_)
            copy_idx(dstSA, dstA)
            pltpu.async_copy(rowsA, acc.at[dstSA], semSA, add=True)

            @pl.when(q + 1 < _NQ)
            def _():
                loads(t0 + 2, srcA, dstA, wbA_, semIA)

            pltpu.make_async_copy(ysrc.at[srcB], rowsB, semGB).wait()
            do_scale(rowsB, wbB_)
            copy_idx(dstSB, dstB)
            pltpu.async_copy(rowsB, acc.at[dstSB], semSB, add=True)

            @pl.when(q + 1 < _NQ)
            def _():
                drain_loads(t0 + 2, srcA, dstA, wbA_, semIA)
                pltpu.make_async_copy(rowsA, acc.at[dstSA], semSA).wait()
                pltpu.async_copy(ysrc.at[srcA], rowsA, semGA)
                loads(t0 + 3, srcB, dstB, wbB_, semIB)

            return carry

        lax.fori_loop(0, _NQ, body, 0)
        pltpu.make_async_copy(rowsA, acc.at[dstSA], semSA).wait()
        pltpu.make_async_copy(rowsB, acc.at[dstSB], semSB).wait()
        plsc.subcore_barrier()
        for j in range(_RPT // 64):
            r0 = sid * _RPT + j * 64
            pltpu.sync_copy(acc.at[pl.ds(r0, 64), :],
                            out_hbm.at[pl.ds(cid * _NPAD + r0, 64), :])

    return k


_spmm_w128 = _make_spmm(_C, True, False)
_spmm_128 = _make_spmm(_C, False, False)
_spmm_16 = _make_spmm(_H1, False, True)


_B = 2000
_G = _N // _B


def _tc_score(x, p2):
    def body(x_ref, p_ref, o_ref):
        pv = p_ref[...]
        nrm = jnp.sqrt(jnp.sum(pv * pv))
        o_ref[...] = jnp.tanh(
            jnp.dot(x_ref[...], pv, preferred_element_type=f32) / nrm)

    return pl.pallas_call(
        body,
        grid=(_G,),
        in_specs=[pl.BlockSpec((_B, _C), lambda i: (i, 0)),
                  pl.BlockSpec((_C, 1), lambda i: (0, 0))],
        out_specs=pl.BlockSpec((_B, 1), lambda i: (i, 0)),
        out_shape=jax.ShapeDtypeStruct((_N, 1), f32),
    )(x, p2)


def _tc_gru(xt, W0, Wih, Whh, bih, bhh):
    def body(xt_ref, w0_ref, wih_ref, whh_ref, bih_ref, bhh_ref, o_ref):
        cdims = (((1,), (1,)), ((), ()))
        gi = lax.dot_general(xt_ref[...], wih_ref[...], cdims,
                             preferred_element_type=f32) + bih_ref[...]
        gh = lax.dot_general(w0_ref[...], whh_ref[...], cdims,
                             preferred_element_type=f32) + bhh_ref[...]
        r = jax.nn.sigmoid(gi[:, :_C] + gh[:, :_C])
        z = jax.nn.sigmoid(gi[:, _C:2 * _C] + gh[:, _C:2 * _C])
        nn_ = jnp.tanh(gi[:, 2 * _C:] + r * gh[:, 2 * _C:])
        o_ref[...] = (1.0 - z) * nn_ + z * w0_ref[...]

    return pl.pallas_call(
        body,
        out_shape=jax.ShapeDtypeStruct((_C, _C), f32),
    )(xt, W0, Wih, Whh, bih, bhh)


def _tc_y1(da0, da1, x, W):
    def body(d0, d1, x_ref, w_ref, o_ref):
        dinv = lax.rsqrt(d0[...] + d1[...] + 1.0)
        o_ref[...] = dinv * jnp.dot(x_ref[...], w_ref[...],
                                    preferred_element_type=f32)

    return pl.pallas_call(
        body,
        grid=(_G,),
        in_specs=[pl.BlockSpec((_B, 1), lambda i: (i, 0)),
                  pl.BlockSpec((_B, 1), lambda i: (i, 0)),
                  pl.BlockSpec((_B, _C), lambda i: (i, 0)),
                  pl.BlockSpec((_C, _C), lambda i: (0, 0))],
        out_specs=pl.BlockSpec((_B, _C), lambda i: (i, 0)),
        out_shape=jax.ShapeDtypeStruct((_N, _C), f32),
    )(da0, da1, x, W)


def _tc_mid2(p0, p1, y1, da0, da1, db0, db1, W1):
    def body(p0r, p1r, y1r, da0r, da1r, db0r, db1r, w1r, o_ref):
        dinva = lax.rsqrt(da0r[...] + da1r[...] + 1.0)
        dinvb = lax.rsqrt(db0r[...] + db1r[...] + 1.0)
        h1 = jnp.maximum(dinva * (p0r[...] + p1r[...] + y1r[...]), 0.0)
        o_ref[...] = dinvb * jnp.dot(h1, w1r[...], preferred_element_type=f32)

    return pl.pallas_call(
        body,
        grid=(_G,),
        in_specs=[pl.BlockSpec((_B, _C), lambda i: (i, 0)),
                  pl.BlockSpec((_B, _C), lambda i: (i, 0)),
                  pl.BlockSpec((_B, _C), lambda i: (i, 0)),
                  pl.BlockSpec((_B, 1), lambda i: (i, 0)),
                  pl.BlockSpec((_B, 1), lambda i: (i, 0)),
                  pl.BlockSpec((_B, 1), lambda i: (i, 0)),
                  pl.BlockSpec((_B, 1), lambda i: (i, 0)),
                  pl.BlockSpec((_C, _H1), lambda i: (0, 0))],
        out_specs=pl.BlockSpec((_B, _H1), lambda i: (i, 0)),
        out_shape=jax.ShapeDtypeStruct((_N, _H1), f32),
    )(p0, p1, y1, da0, da1, db0, db1, W1)


def _tc_mid3(p0, p1, y2, db0, db1, dc0, dc1, W2, b1):
    def body(p0r, p1r, y2r, db0r, db1r, dc0r, dc1r, w2r, b1r, o_ref):
        dinvb = lax.rsqrt(db0r[...] + db1r[...] + 1.0)
        dinvc = lax.rsqrt(dc0r[...] + dc1r[...] + 1.0)
        h2 = jnp.maximum(
            dinvb * (p0r[...] + p1r[...] + y2r[...]) + b1r[...], 0.0)
        o_ref[...] = dinvc * jnp.dot(h2, w2r[...], preferred_element_type=f32)

    return pl.pallas_call(
        body,
        grid=(_G,),
        in_specs=[pl.BlockSpec((_B, _H1), lambda i: (i, 0)),
                  pl.BlockSpec((_B, _H1), lambda i: (i, 0)),
                  pl.BlockSpec((_B, _H1), lambda i: (i, 0)),
                  pl.BlockSpec((_B, 1), lambda i: (i, 0)),
                  pl.BlockSpec((_B, 1), lambda i: (i, 0)),
                  pl.BlockSpec((_B, 1), lambda i: (i, 0)),
                  pl.BlockSpec((_B, 1), lambda i: (i, 0)),
                  pl.BlockSpec((_H1, _C), lambda i: (0, 0)),
                  pl.BlockSpec((1, _H1), lambda i: (0, 0))],
        out_specs=pl.BlockSpec((_B, _C), lambda i: (i, 0)),
        out_shape=jax.ShapeDtypeStruct((_N, _C), f32),
    )(p0, p1, y2, db0, db1, dc0, dc1, W2, b1)


def _tc_fin(p0, p1, y3, dc0, dc1, b2):
    def body(p0r, p1r, y3r, dc0r, dc1r, b2r, o_ref):
        dinvc = lax.rsqrt(dc0r[...] + dc1r[...] + 1.0)
        o_ref[...] = dinvc * (p0r[...] + p1r[...] + y3r[...]) + b2r[...]

    return pl.pallas_call(
        body,
        grid=(_G,),
        in_specs=[pl.BlockSpec((_B, _C), lambda i: (i, 0)),
                  pl.BlockSpec((_B, _C), lambda i: (i, 0)),
                  pl.BlockSpec((_B, _C), lambda i: (i, 0)),
                  pl.BlockSpec((_B, 1), lambda i: (i, 0)),
                  pl.BlockSpec((_B, 1), lambda i: (i, 0)),
                  pl.BlockSpec((1, _C), lambda i: (0, 0))],
        out_specs=pl.BlockSpec((_B, _C), lambda i: (i, 0)),
        out_shape=jax.ShapeDtypeStruct((_N, _C), f32),
    )(p0, p1, y3, dc0, dc1, b2)


def kernel(x, edge_index, edge_weight, tR_indices, p,
           W_ih, W_hh, b_ih, b_hh, W0, W1, b1, W2, b2):
    npad = _EPAD - _E
    pad_src = (jnp.arange(npad, dtype=i32) * 53) % _N
    pad_dst = _N + (jnp.arange(npad, dtype=i32) % (_NPAD - _N))
    pad_blk = jnp.stack([pad_src, pad_dst])
    eip = jnp.concatenate([edge_index.astype(i32), pad_blk], axis=1)
    trp = jnp.concatenate([tR_indices.astype(i32), pad_blk], axis=1)
    wp = jnp.concatenate([edge_weight, jnp.zeros((npad,), f32)])

    degw = _sc_deg(eip, trp, wp).reshape(_NC, 3, _NPAD // 8, 128)

    def _dslice(c, a):
        return degw[c, a, :, 0::16].reshape(_NPAD, 1)[:_N]

    da0, da1 = _dslice(0, 0), _dslice(1, 0)
    db0, db1 = _dslice(0, 1), _dslice(1, 1)
    dc0, dc1 = _dslice(0, 2), _dslice(1, 2)

    score = _tc_score(x, p.reshape(_C, 1))[:, 0]
    vals, idx = lax.top_k(score, _C)
    x_tilde = x[idx] * vals[:, None]
    Wt = _tc_gru(x_tilde, W0, W_ih, W_hh,
                 b_ih.reshape(1, -1), b_hh.reshape(1, -1))

    y1 = _tc_y1(da0, da1, x, Wt)
    p1 = _spmm_w128(eip, wp, y1).reshape(_NC, _NPAD, _C)
    y2 = _tc_mid2(p1[0, :_N], p1[1, :_N], y1, da0, da1, db0, db1, W1)
    p2 = _spmm_16(eip, y2).reshape(_NC, _NPAD, _H1)
    y3 = _tc_mid3(p2[0, :_N], p2[1, :_N], y2, db0, db1, dc0, dc1,
                  W2, b1.reshape(1, -1))
    p3 = _spmm_128(trp, y3).reshape(_NC, _NPAD, _C)
    return _tc_fin(p3[0, :_N], p3[1, :_N], y3, dc0, dc1, b2.reshape(1, -1))
